# Initial kernel scaffold; baseline (speedup 1.0000x reference)
import jax
import jax.numpy as jnp
from jax.experimental import pallas as pl


def kernel(x_features, camera_cond, W1, b1, gamma1, beta1, W2, b2, gamma2, beta2, Wc, bc, edge_index, kernel_offsets, batch_idx):
    raise NotImplementedError("write your pallas kernel here")


# speedup vs baseline: 1.7967x; 1.7967x over previous
import functools

import jax
import jax.numpy as jnp
from jax import lax
from jax.experimental import pallas as pl
from jax.experimental.pallas import tpu as pltpu
from jax.experimental.pallas import tpu_sc as plsc

_N = 10000
_E = 320000
_C = 128
_K = 27
_CAM = 256
_EPS = 1e-5

_NSC = 2
_NSUB = 16
_NT = _NSC * _NSUB
_EP = _E // _NT
_CH = 80
_NCH = _EP // _CH
_NPAD = 10240
_RPT = _NPAD // _NSUB
_RCH = 80
_NB = 25
_R = _N // _NB


def _dense_body(apply_bn, x_ref, w_ref, *rest):
    k = pl.program_id(1)
    if apply_bn:
        s_ref, q_ref, g_ref, b_ref, y_ref, hn_ref = rest

        @pl.when(k == 0)
        def _():
            inv_n = jnp.float32(1.0 / _N)
            mu = s_ref[...] * inv_n
            var = q_ref[...] * inv_n - mu * mu
            hn = g_ref[...] * (x_ref[...] - mu) * lax.rsqrt(var + _EPS)
            hn_ref[...] = jnp.maximum(hn + b_ref[...], 0.0)

        xb = hn_ref[...]
    else:
        (y_ref,) = rest
        xb = x_ref[...]
    y_ref[...] = jnp.dot(xb, w_ref[k], preferred_element_type=jnp.float32)


def _dense_stage(x, W, stats=None):
    apply_bn = stats is not None
    in_specs = [
        pl.BlockSpec((_R, _C), lambda i, k: (i, 0)),
        pl.BlockSpec((_K, _C, _C), lambda i, k: (0, 0, 0)),
    ]
    args = [x, W]
    scratch = []
    if apply_bn:
        in_specs += [pl.BlockSpec((1, _C), lambda i, k: (0, 0))] * 4
        args += list(stats)
        scratch = [pltpu.VMEM((_R, _C), jnp.float32)]
    return pl.pallas_call(
        functools.partial(_dense_body, apply_bn),
        grid=(_NB, _K),
        in_specs=in_specs,
        out_specs=pl.BlockSpec((_R, _C), lambda i, k: (k * _NB + i, 0)),
        out_shape=jax.ShapeDtypeStruct((_K * _N, _C), jnp.float32),
        scratch_shapes=scratch,
        compiler_params=pltpu.CompilerParams(
            dimension_semantics=("arbitrary", "arbitrary")),
    )(*args)


def _prep_body(src_ref, koff_ref, g_ref):
    g_ref[...] = koff_ref[...] * _N + src_ref[...]


def _prep_stage(src, koff):
    s2 = src.reshape(_E // 128, 128)
    k2 = koff.reshape(_E // 128, 128)
    g2 = pl.pallas_call(
        _prep_body,
        out_shape=jax.ShapeDtypeStruct((_E // 128, 128), jnp.int32),
    )(s2, k2)
    return g2.reshape(_E)


def _sc_scatter(table, g, dst):
    mesh = plsc.VectorSubcoreMesh(core_axis_name="c", subcore_axis_name="s")

    @functools.partial(
        pl.kernel,
        out_type=jax.ShapeDtypeStruct((_NSC * _NPAD, _C), jnp.float32),
        mesh=mesh,
        scratch_types=[
            pltpu.VMEM((_EP,), jnp.int32),
            pltpu.VMEM((_EP,), jnp.int32),
            pltpu.VMEM((_CH,), jnp.int32),
            pltpu.VMEM((_CH, _C), jnp.float32),
            pltpu.VMEM((8, _C), jnp.float32),
            pltpu.VMEM_SHARED((_NPAD, _C), jnp.float32),
            pltpu.SemaphoreType.DMA,
        ],
    )
    def sc_kernel(table_h, g_h, dst_h, out_h,
                  g_v, d_v, db_v, rows_v, zb_v, acc_s, sem):
        cid = lax.axis_index("c")
        sid = lax.axis_index("s")
        wid = sid * _NSC + cid
        ebase = pl.multiple_of(wid * _EP, 8)

        pltpu.sync_copy(g_h.at[pl.ds(ebase, _EP)], g_v)
        pltpu.sync_copy(dst_h.at[pl.ds(ebase, _EP)], d_v)

        def zvec(i, c):
            def zlane(j, c2):
                zb_v[i, pl.ds(j * 16, 16)] = jnp.zeros((16,), jnp.float32)
                return c2
            return lax.fori_loop(0, _C // 16, zlane, c)
        lax.fori_loop(0, 8, zvec, 0)

        rbase = sid * _RPT

        def zrow(j, c):
            pltpu.sync_copy(zb_v, acc_s.at[pl.ds(rbase + j * 8, 8)])
            return c
        lax.fori_loop(0, _RPT // 8, zrow, 0)
        plsc.subcore_barrier()

        def chunk(i, c):
            eb = pl.multiple_of(i * _CH, 8)

            def cp(j, c2):
                o = j * 16
                db_v[pl.ds(o, 16)] = d_v[pl.ds(eb + o, 16)]
                return c2
            lax.fori_loop(0, _CH // 16, cp, 0)
            pltpu.async_copy(table_h.at[g_v.at[pl.ds(eb, _CH)]], rows_v,
                             sem).wait()
            pltpu.sync_copy(rows_v, acc_s.at[db_v], add=True)
            return c
        lax.fori_loop(0, _NCH, chunk, 0)
        plsc.subcore_barrier()

        obase = cid * _NPAD + rbase

        def drain(j, c):
            pltpu.sync_copy(acc_s.at[pl.ds(rbase + j * 8, 8)], zb_v)
            pltpu.sync_copy(zb_v, out_h.at[pl.ds(obase + j * 8, 8)])
            return c
        lax.fori_loop(0, _RPT // 8, drain, 0)

    return sc_kernel(table, g, dst)


def _stats_body(p_ref, h_ref, sum_ref, sq_ref):
    i = pl.program_id(0)
    h = p_ref[0] + p_ref[1]
    h_ref[...] = h
    s = jnp.sum(h, axis=0, keepdims=True)
    q = jnp.sum(h * h, axis=0, keepdims=True)

    @pl.when(i == 0)
    def _():
        sum_ref[...] = s
        sq_ref[...] = q

    @pl.when(i > 0)
    def _():
        sum_ref[...] = sum_ref[...] + s
        sq_ref[...] = sq_ref[...] + q


def _stats_stage(partials):
    return pl.pallas_call(
        _stats_body,
        grid=(_NB,),
        in_specs=[pl.BlockSpec((_NSC, _R, _C), lambda i: (0, i, 0))],
        out_specs=[
            pl.BlockSpec((_R, _C), lambda i: (i, 0)),
            pl.BlockSpec((1, _C), lambda i: (0, 0)),
            pl.BlockSpec((1, _C), lambda i: (0, 0)),
        ],
        out_shape=[
            jax.ShapeDtypeStruct((_N, _C), jnp.float32),
            jax.ShapeDtypeStruct((1, _C), jnp.float32),
            jax.ShapeDtypeStruct((1, _C), jnp.float32),
        ],
        compiler_params=pltpu.CompilerParams(
            dimension_semantics=("arbitrary",)),
    )(partials)


def _final_body(h_ref, s_ref, q_ref, g_ref, b_ref, cam_ref, wc_ref, bc_ref,
                bidx_ref, x_ref, o_ref):
    inv_n = jnp.float32(1.0 / _N)
    mu = s_ref[...] * inv_n
    var = q_ref[...] * inv_n - mu * mu
    hn = g_ref[...] * (h_ref[...] - mu) * lax.rsqrt(var + _EPS) + b_ref[...]
    cam = jnp.dot(cam_ref[...], wc_ref[...],
                  preferred_element_type=jnp.float32) + bc_ref[...]
    bi = bidx_ref[0, 0, :]
    onehot = (bi[:, None] == lax.broadcasted_iota(jnp.int32, (1, 8), 1)
              ).astype(jnp.float32)
    film = jnp.dot(onehot, cam, preferred_element_type=jnp.float32)
    scale = film[:, :_C]
    shift = film[:, _C:]
    o_ref[...] = jnp.maximum(hn * (1.0 + scale) + shift, 0.0) + x_ref[...]


def _final_stage(h, s, q, gamma, beta, cam_pad, Wc, bc, bidx3, x):
    return pl.pallas_call(
        _final_body,
        grid=(_NB,),
        in_specs=[
            pl.BlockSpec((_R, _C), lambda i: (i, 0)),
            pl.BlockSpec((1, _C), lambda i: (0, 0)),
            pl.BlockSpec((1, _C), lambda i: (0, 0)),
            pl.BlockSpec((1, _C), lambda i: (0, 0)),
            pl.BlockSpec((1, _C), lambda i: (0, 0)),
            pl.BlockSpec((8, 2 * _C), lambda i: (0, 0)),
            pl.BlockSpec((_CAM, 2 * _C), lambda i: (0, 0)),
            pl.BlockSpec((1, 2 * _C), lambda i: (0, 0)),
            pl.BlockSpec((1, 1, _R), lambda i: (i, 0, 0)),
            pl.BlockSpec((_R, _C), lambda i: (i, 0)),
        ],
        out_specs=pl.BlockSpec((_R, _C), lambda i: (i, 0)),
        out_shape=jax.ShapeDtypeStruct((_N, _C), jnp.float32),
        compiler_params=pltpu.CompilerParams(
            dimension_semantics=("arbitrary",)),
    )(h, s, q, gamma, beta, cam_pad, Wc, bc, bidx3, x)


def kernel(x_features, camera_cond, W1, b1, gamma1, beta1, W2, b2, gamma2,
           beta2, Wc, bc, edge_index, kernel_offsets, batch_idx):
    del b1, b2
    src = edge_index[0]
    dst = edge_index[1]
    g = _prep_stage(src, kernel_offsets)

    y1 = _dense_stage(x_features, W1)
    p1 = _sc_scatter(y1, g, dst)
    h1, s1, q1 = _stats_stage(p1.reshape(_NSC, _NPAD, _C))

    y2 = _dense_stage(h1, W2, stats=(s1, q1, gamma1.reshape(1, _C),
                                     beta1.reshape(1, _C)))
    p2 = _sc_scatter(y2, g, dst)
    h2, s2, q2 = _stats_stage(p2.reshape(_NSC, _NPAD, _C))

    cam_pad = jnp.zeros((8, _CAM), jnp.float32).at[:4].set(camera_cond)
    bidx3 = batch_idx.reshape(_NB, 1, _R)
    return _final_stage(h2, s2, q2, gamma2.reshape(1, _C),
                        beta2.reshape(1, _C), cam_pad, Wc,
                        bc.reshape(1, 2 * _C), bidx3, x_features)


# speedup vs baseline: 1.9586x; 1.0901x over previous
import functools

import jax
import jax.numpy as jnp
from jax import lax
from jax.experimental import pallas as pl
from jax.experimental.pallas import tpu as pltpu
from jax.experimental.pallas import tpu_sc as plsc

_N = 10000
_E = 320000
_C = 128
_K = 27
_CAM = 256
_EPS = 1e-5

_NSC = 2
_NSUB = 16
_NT = _NSC * _NSUB
_EP = _E // _NT
_CH = 80
_NCH = _EP // _CH
_NPAD = 10240
_RPT = _NPAD // _NSUB
_RCH = 80
_NB = 25
_R = _N // _NB


def _dense_body(apply_bn, x_ref, w_ref, *rest):
    k = pl.program_id(1)
    if apply_bn:
        s_ref, q_ref, g_ref, b_ref, y_ref, hn_ref = rest

        @pl.when(k == 0)
        def _():
            inv_n = jnp.float32(1.0 / _N)
            mu = s_ref[...] * inv_n
            var = q_ref[...] * inv_n - mu * mu
            hn = g_ref[...] * (x_ref[...] - mu) * lax.rsqrt(var + _EPS)
            hn_ref[...] = jnp.maximum(hn + b_ref[...], 0.0)

        xb = hn_ref[...]
    else:
        (y_ref,) = rest
        xb = x_ref[...]
    y_ref[...] = jnp.dot(xb, w_ref[k], preferred_element_type=jnp.float32)


def _dense_stage(x, W, stats=None):
    apply_bn = stats is not None
    in_specs = [
        pl.BlockSpec((_R, _C), lambda i, k: (i, 0)),
        pl.BlockSpec((_K, _C, _C), lambda i, k: (0, 0, 0)),
    ]
    args = [x, W]
    scratch = []
    if apply_bn:
        in_specs += [pl.BlockSpec((1, _C), lambda i, k: (0, 0))] * 4
        args += list(stats)
        scratch = [pltpu.VMEM((_R, _C), jnp.float32)]
    return pl.pallas_call(
        functools.partial(_dense_body, apply_bn),
        grid=(_NB, _K),
        in_specs=in_specs,
        out_specs=pl.BlockSpec((_R, _C), lambda i, k: (k * _NB + i, 0)),
        out_shape=jax.ShapeDtypeStruct((_K * _N, _C), jnp.float32),
        scratch_shapes=scratch,
        compiler_params=pltpu.CompilerParams(
            dimension_semantics=("arbitrary", "arbitrary")),
    )(*args)


def _prep_body(src_ref, koff_ref, g_ref):
    g_ref[...] = koff_ref[...] * _N + src_ref[...]


def _prep_stage(src, koff):
    s2 = src.reshape(_E // 128, 128)
    k2 = koff.reshape(_E // 128, 128)
    g2 = pl.pallas_call(
        _prep_body,
        out_shape=jax.ShapeDtypeStruct((_E // 128, 128), jnp.int32),
    )(s2, k2)
    return g2.reshape(_E)


def _sc_scatter(table, g, dst):
    mesh = plsc.VectorSubcoreMesh(core_axis_name="c", subcore_axis_name="s")

    @functools.partial(
        pl.kernel,
        out_type=jax.ShapeDtypeStruct((_NSC * _NPAD, _C), jnp.float32),
        mesh=mesh,
        scratch_types=[
            pltpu.VMEM((_EP,), jnp.int32),
            pltpu.VMEM((_EP,), jnp.int32),
            pltpu.VMEM((_CH,), jnp.int32),
            pltpu.VMEM((_CH,), jnp.int32),
            pltpu.VMEM((_CH, _C), jnp.float32),
            pltpu.VMEM((_CH, _C), jnp.float32),
            pltpu.VMEM((8, _C), jnp.float32),
            pltpu.VMEM_SHARED((_NPAD, _C), jnp.float32),
            pltpu.SemaphoreType.DMA,
            pltpu.SemaphoreType.DMA,
        ],
    )
    def sc_kernel(table_h, g_h, dst_h, out_h,
                  g_v, d_v, db0_v, db1_v, rows0_v, rows1_v, zb_v, acc_s,
                  sem0, sem1):
        cid = lax.axis_index("c")
        sid = lax.axis_index("s")
        wid = sid * _NSC + cid
        ebase = pl.multiple_of(wid * _EP, 8)

        pltpu.sync_copy(g_h.at[pl.ds(ebase, _EP)], g_v)
        pltpu.sync_copy(dst_h.at[pl.ds(ebase, _EP)], d_v)

        def zvec(i, c):
            def zlane(j, c2):
                zb_v[i, pl.ds(j * 16, 16)] = jnp.zeros((16,), jnp.float32)
                return c2
            return lax.fori_loop(0, _C // 16, zlane, c)
        lax.fori_loop(0, 8, zvec, 0)

        rbase = sid * _RPT

        def zrow(j, c):
            pltpu.sync_copy(zb_v, acc_s.at[pl.ds(rbase + j * 8, 8)])
            return c
        lax.fori_loop(0, _RPT // 8, zrow, 0)
        plsc.subcore_barrier()

        def cp(eb, db):
            def cpb(j, c2):
                o = j * 16
                db[pl.ds(o, 16)] = d_v[pl.ds(eb + o, 16)]
                return c2
            lax.fori_loop(0, _CH // 16, cpb, 0)

        def pair(p, c):
            eb0 = pl.multiple_of(p * (2 * _CH), 8)
            eb1 = pl.multiple_of(p * (2 * _CH) + _CH, 8)
            cp(eb0, db0_v)
            cpy0 = pltpu.async_copy(table_h.at[g_v.at[pl.ds(eb0, _CH)]],
                                    rows0_v, sem0)
            cp(eb1, db1_v)
            cpy1 = pltpu.async_copy(table_h.at[g_v.at[pl.ds(eb1, _CH)]],
                                    rows1_v, sem1)
            cpy0.wait()
            pltpu.sync_copy(rows0_v, acc_s.at[db0_v], add=True)
            cpy1.wait()
            pltpu.sync_copy(rows1_v, acc_s.at[db1_v], add=True)
            return c
        lax.fori_loop(0, _NCH // 2, pair, 0)

        ebt = pl.multiple_of((_NCH - 1) * _CH, 8)
        cp(ebt, db0_v)
        pltpu.async_copy(table_h.at[g_v.at[pl.ds(ebt, _CH)]], rows0_v,
                         sem0).wait()
        pltpu.sync_copy(rows0_v, acc_s.at[db0_v], add=True)
        plsc.subcore_barrier()

        obase = cid * _NPAD + rbase

        def drain(j, c):
            pltpu.sync_copy(acc_s.at[pl.ds(rbase + j * 8, 8)], zb_v)
            pltpu.sync_copy(zb_v, out_h.at[pl.ds(obase + j * 8, 8)])
            return c
        lax.fori_loop(0, _RPT // 8, drain, 0)

    return sc_kernel(table, g, dst)


def _stats_body(p_ref, h_ref, sum_ref, sq_ref):
    i = pl.program_id(0)
    h = p_ref[0] + p_ref[1]
    h_ref[...] = h
    s = jnp.sum(h, axis=0, keepdims=True)
    q = jnp.sum(h * h, axis=0, keepdims=True)

    @pl.when(i == 0)
    def _():
        sum_ref[...] = s
        sq_ref[...] = q

    @pl.when(i > 0)
    def _():
        sum_ref[...] = sum_ref[...] + s
        sq_ref[...] = sq_ref[...] + q


def _stats_stage(partials):
    return pl.pallas_call(
        _stats_body,
        grid=(_NB,),
        in_specs=[pl.BlockSpec((_NSC, _R, _C), lambda i: (0, i, 0))],
        out_specs=[
            pl.BlockSpec((_R, _C), lambda i: (i, 0)),
            pl.BlockSpec((1, _C), lambda i: (0, 0)),
            pl.BlockSpec((1, _C), lambda i: (0, 0)),
        ],
        out_shape=[
            jax.ShapeDtypeStruct((_N, _C), jnp.float32),
            jax.ShapeDtypeStruct((1, _C), jnp.float32),
            jax.ShapeDtypeStruct((1, _C), jnp.float32),
        ],
        compiler_params=pltpu.CompilerParams(
            dimension_semantics=("arbitrary",)),
    )(partials)


def _final_body(h_ref, s_ref, q_ref, g_ref, b_ref, cam_ref, wc_ref, bc_ref,
                bidx_ref, x_ref, o_ref):
    inv_n = jnp.float32(1.0 / _N)
    mu = s_ref[...] * inv_n
    var = q_ref[...] * inv_n - mu * mu
    hn = g_ref[...] * (h_ref[...] - mu) * lax.rsqrt(var + _EPS) + b_ref[...]
    cam = jnp.dot(cam_ref[...], wc_ref[...],
                  preferred_element_type=jnp.float32) + bc_ref[...]
    bi = bidx_ref[0, 0, :]
    onehot = (bi[:, None] == lax.broadcasted_iota(jnp.int32, (1, 8), 1)
              ).astype(jnp.float32)
    film = jnp.dot(onehot, cam, preferred_element_type=jnp.float32)
    scale = film[:, :_C]
    shift = film[:, _C:]
    o_ref[...] = jnp.maximum(hn * (1.0 + scale) + shift, 0.0) + x_ref[...]


def _final_stage(h, s, q, gamma, beta, cam_pad, Wc, bc, bidx3, x):
    return pl.pallas_call(
        _final_body,
        grid=(_NB,),
        in_specs=[
            pl.BlockSpec((_R, _C), lambda i: (i, 0)),
            pl.BlockSpec((1, _C), lambda i: (0, 0)),
            pl.BlockSpec((1, _C), lambda i: (0, 0)),
            pl.BlockSpec((1, _C), lambda i: (0, 0)),
            pl.BlockSpec((1, _C), lambda i: (0, 0)),
            pl.BlockSpec((8, 2 * _C), lambda i: (0, 0)),
            pl.BlockSpec((_CAM, 2 * _C), lambda i: (0, 0)),
            pl.BlockSpec((1, 2 * _C), lambda i: (0, 0)),
            pl.BlockSpec((1, 1, _R), lambda i: (i, 0, 0)),
            pl.BlockSpec((_R, _C), lambda i: (i, 0)),
        ],
        out_specs=pl.BlockSpec((_R, _C), lambda i: (i, 0)),
        out_shape=jax.ShapeDtypeStruct((_N, _C), jnp.float32),
        compiler_params=pltpu.CompilerParams(
            dimension_semantics=("arbitrary",)),
    )(h, s, q, gamma, beta, cam_pad, Wc, bc, bidx3, x)


def kernel(x_features, camera_cond, W1, b1, gamma1, beta1, W2, b2, gamma2,
           beta2, Wc, bc, edge_index, kernel_offsets, batch_idx):
    del b1, b2
    src = edge_index[0]
    dst = edge_index[1]
    g = _prep_stage(src, kernel_offsets)

    y1 = _dense_stage(x_features, W1)
    p1 = _sc_scatter(y1, g, dst)
    h1, s1, q1 = _stats_stage(p1.reshape(_NSC, _NPAD, _C))

    y2 = _dense_stage(h1, W2, stats=(s1, q1, gamma1.reshape(1, _C),
                                     beta1.reshape(1, _C)))
    p2 = _sc_scatter(y2, g, dst)
    h2, s2, q2 = _stats_stage(p2.reshape(_NSC, _NPAD, _C))

    cam_pad = jnp.zeros((8, _CAM), jnp.float32).at[:4].set(camera_cond)
    bidx3 = batch_idx.reshape(_NB, 1, _R)
    return _final_stage(h2, s2, q2, gamma2.reshape(1, _C),
                        beta2.reshape(1, _C), cam_pad, Wc,
                        bc.reshape(1, 2 * _C), bidx3, x_features)


# speedup vs baseline: 2.5879x; 1.3213x over previous
import functools

import jax
import jax.numpy as jnp
from jax import lax
from jax.experimental import pallas as pl
from jax.experimental.pallas import tpu as pltpu
from jax.experimental.pallas import tpu_sc as plsc

_N = 10000
_E = 320000
_C = 128
_K = 27
_CAM = 256
_EPS = 1e-5

_NSC = 2
_NSUB = 16
_NT = _NSC * _NSUB
_EP = _E // _NT
_CH = 80
_NCH = _EP // _CH
_NPAD = 10240
_RPT = _NPAD // _NSUB
_RCH = 80
_NB = 25
_R = _N // _NB


def _dense_body(apply_bn, x_ref, w_ref, *rest):
    if apply_bn:
        s_ref, q_ref, g_ref, b_ref, y_ref = rest
        inv_n = jnp.float32(1.0 / _N)
        mu = s_ref[...] * inv_n
        var = q_ref[...] * inv_n - mu * mu
        hn = g_ref[...] * (x_ref[...] - mu) * lax.rsqrt(var + _EPS)
        xb = jnp.maximum(hn + b_ref[...], 0.0)
    else:
        (y_ref,) = rest
        xb = x_ref[...]
    y_ref[...] = jnp.dot(xb.astype(jnp.bfloat16), w_ref[...],
                         preferred_element_type=jnp.float32)


def _dense_stage(x, Wwide, stats=None):
    apply_bn = stats is not None
    in_specs = [
        pl.BlockSpec((_R, _C), lambda i: (i, 0)),
        pl.BlockSpec((_C, _K * _C), lambda i: (0, 0)),
    ]
    args = [x, Wwide]
    if apply_bn:
        in_specs += [pl.BlockSpec((1, _C), lambda i: (0, 0))] * 4
        args += list(stats)
    return pl.pallas_call(
        functools.partial(_dense_body, apply_bn),
        grid=(_NB,),
        in_specs=in_specs,
        out_specs=pl.BlockSpec((_R, _K * _C), lambda i: (i, 0)),
        out_shape=jax.ShapeDtypeStruct((_N, _K * _C), jnp.float32),
        compiler_params=pltpu.CompilerParams(
            dimension_semantics=("arbitrary",)),
    )(*args)


def _prep_body(src_ref, koff_ref, g_ref):
    g_ref[...] = src_ref[...] * _K + koff_ref[...]


def _prep_stage(src, koff):
    s2 = src.reshape(_E // 128, 128)
    k2 = koff.reshape(_E // 128, 128)
    g2 = pl.pallas_call(
        _prep_body,
        out_shape=jax.ShapeDtypeStruct((_E // 128, 128), jnp.int32),
    )(s2, k2)
    return g2.reshape(_E)


def _sc_scatter(table, g, dst):
    mesh = plsc.VectorSubcoreMesh(core_axis_name="c", subcore_axis_name="s")

    @functools.partial(
        pl.kernel,
        out_type=jax.ShapeDtypeStruct((_NSC * _NPAD, _C), jnp.float32),
        mesh=mesh,
        scratch_types=[
            pltpu.VMEM((_EP,), jnp.int32),
            pltpu.VMEM((_EP,), jnp.int32),
            pltpu.VMEM((_CH,), jnp.int32),
            pltpu.VMEM((_CH,), jnp.int32),
            pltpu.VMEM((_CH, _C), jnp.float32),
            pltpu.VMEM((_CH, _C), jnp.float32),
            pltpu.VMEM((8, _C), jnp.float32),
            pltpu.VMEM_SHARED((_NPAD, _C), jnp.float32),
            pltpu.SemaphoreType.DMA,
            pltpu.SemaphoreType.DMA,
        ],
    )
    def sc_kernel(table_h, g_h, dst_h, out_h,
                  g_v, d_v, db0_v, db1_v, rows0_v, rows1_v, zb_v, acc_s,
                  sem0, sem1):
        cid = lax.axis_index("c")
        sid = lax.axis_index("s")
        wid = sid * _NSC + cid
        ebase = pl.multiple_of(wid * _EP, 8)

        pltpu.sync_copy(g_h.at[pl.ds(ebase, _EP)], g_v)
        pltpu.sync_copy(dst_h.at[pl.ds(ebase, _EP)], d_v)

        def zvec(i, c):
            def zlane(j, c2):
                zb_v[i, pl.ds(j * 16, 16)] = jnp.zeros((16,), jnp.float32)
                return c2
            return lax.fori_loop(0, _C // 16, zlane, c)
        lax.fori_loop(0, 8, zvec, 0)

        rbase = sid * _RPT

        def zrow(j, c):
            pltpu.sync_copy(zb_v, acc_s.at[pl.ds(rbase + j * 8, 8)])
            return c
        lax.fori_loop(0, _RPT // 8, zrow, 0)
        plsc.subcore_barrier()

        def cp(eb, db):
            def cpb(j, c2):
                o = j * 16
                db[pl.ds(o, 16)] = d_v[pl.ds(eb + o, 16)]
                return c2
            lax.fori_loop(0, _CH // 16, cpb, 0)

        def pair(p, c):
            eb0 = pl.multiple_of(p * (2 * _CH), 8)
            eb1 = pl.multiple_of(p * (2 * _CH) + _CH, 8)
            cp(eb0, db0_v)
            cpy0 = pltpu.async_copy(table_h.at[g_v.at[pl.ds(eb0, _CH)]],
                                    rows0_v, sem0)
            cp(eb1, db1_v)
            cpy1 = pltpu.async_copy(table_h.at[g_v.at[pl.ds(eb1, _CH)]],
                                    rows1_v, sem1)
            cpy0.wait()
            pltpu.sync_copy(rows0_v, acc_s.at[db0_v], add=True)
            cpy1.wait()
            pltpu.sync_copy(rows1_v, acc_s.at[db1_v], add=True)
            return c
        lax.fori_loop(0, _NCH // 2, pair, 0)

        ebt = pl.multiple_of((_NCH - 1) * _CH, 8)
        cp(ebt, db0_v)
        pltpu.async_copy(table_h.at[g_v.at[pl.ds(ebt, _CH)]], rows0_v,
                         sem0).wait()
        pltpu.sync_copy(rows0_v, acc_s.at[db0_v], add=True)
        plsc.subcore_barrier()

        obase = cid * _NPAD + rbase

        def drain(j, c):
            pltpu.sync_copy(acc_s.at[pl.ds(rbase + j * 8, 8)], zb_v)
            pltpu.sync_copy(zb_v, out_h.at[pl.ds(obase + j * 8, 8)])
            return c
        lax.fori_loop(0, _RPT // 8, drain, 0)

    return sc_kernel(table, g, dst)


def _stats_body(p_ref, h_ref, sum_ref, sq_ref):
    i = pl.program_id(0)
    h = p_ref[0] + p_ref[1]
    h_ref[...] = h
    s = jnp.sum(h, axis=0, keepdims=True)
    q = jnp.sum(h * h, axis=0, keepdims=True)

    @pl.when(i == 0)
    def _():
        sum_ref[...] = s
        sq_ref[...] = q

    @pl.when(i > 0)
    def _():
        sum_ref[...] = sum_ref[...] + s
        sq_ref[...] = sq_ref[...] + q


def _stats_stage(partials):
    return pl.pallas_call(
        _stats_body,
        grid=(_NB,),
        in_specs=[pl.BlockSpec((_NSC, _R, _C), lambda i: (0, i, 0))],
        out_specs=[
            pl.BlockSpec((_R, _C), lambda i: (i, 0)),
            pl.BlockSpec((1, _C), lambda i: (0, 0)),
            pl.BlockSpec((1, _C), lambda i: (0, 0)),
        ],
        out_shape=[
            jax.ShapeDtypeStruct((_N, _C), jnp.float32),
            jax.ShapeDtypeStruct((1, _C), jnp.float32),
            jax.ShapeDtypeStruct((1, _C), jnp.float32),
        ],
        compiler_params=pltpu.CompilerParams(
            dimension_semantics=("arbitrary",)),
    )(partials)


def _final_body(h_ref, s_ref, q_ref, g_ref, b_ref, cam_ref, wc_ref, bc_ref,
                bidx_ref, x_ref, o_ref):
    inv_n = jnp.float32(1.0 / _N)
    mu = s_ref[...] * inv_n
    var = q_ref[...] * inv_n - mu * mu
    hn = g_ref[...] * (h_ref[...] - mu) * lax.rsqrt(var + _EPS) + b_ref[...]
    cam = jnp.dot(cam_ref[...], wc_ref[...],
                  preferred_element_type=jnp.float32) + bc_ref[...]
    bi = bidx_ref[0, 0, :]
    onehot = (bi[:, None] == lax.broadcasted_iota(jnp.int32, (1, 8), 1)
              ).astype(jnp.float32)
    film = jnp.dot(onehot, cam, preferred_element_type=jnp.float32)
    scale = film[:, :_C]
    shift = film[:, _C:]
    o_ref[...] = jnp.maximum(hn * (1.0 + scale) + shift, 0.0) + x_ref[...]


def _final_stage(h, s, q, gamma, beta, cam_pad, Wc, bc, bidx3, x):
    return pl.pallas_call(
        _final_body,
        grid=(_NB,),
        in_specs=[
            pl.BlockSpec((_R, _C), lambda i: (i, 0)),
            pl.BlockSpec((1, _C), lambda i: (0, 0)),
            pl.BlockSpec((1, _C), lambda i: (0, 0)),
            pl.BlockSpec((1, _C), lambda i: (0, 0)),
            pl.BlockSpec((1, _C), lambda i: (0, 0)),
            pl.BlockSpec((8, 2 * _C), lambda i: (0, 0)),
            pl.BlockSpec((_CAM, 2 * _C), lambda i: (0, 0)),
            pl.BlockSpec((1, 2 * _C), lambda i: (0, 0)),
            pl.BlockSpec((1, 1, _R), lambda i: (i, 0, 0)),
            pl.BlockSpec((_R, _C), lambda i: (i, 0)),
        ],
        out_specs=pl.BlockSpec((_R, _C), lambda i: (i, 0)),
        out_shape=jax.ShapeDtypeStruct((_N, _C), jnp.float32),
        compiler_params=pltpu.CompilerParams(
            dimension_semantics=("arbitrary",)),
    )(h, s, q, gamma, beta, cam_pad, Wc, bc, bidx3, x)


def kernel(x_features, camera_cond, W1, b1, gamma1, beta1, W2, b2, gamma2,
           beta2, Wc, bc, edge_index, kernel_offsets, batch_idx):
    del b1, b2
    src = edge_index[0]
    dst = edge_index[1]
    g = _prep_stage(src, kernel_offsets)
    W1w = W1.transpose(1, 0, 2).reshape(_C, _K * _C).astype(jnp.bfloat16)
    W2w = W2.transpose(1, 0, 2).reshape(_C, _K * _C).astype(jnp.bfloat16)

    y1 = _dense_stage(x_features, W1w)
    p1 = _sc_scatter(y1.reshape(_N * _K, _C), g, dst)
    h1, s1, q1 = _stats_stage(p1.reshape(_NSC, _NPAD, _C))

    y2 = _dense_stage(h1, W2w, stats=(s1, q1, gamma1.reshape(1, _C),
                                      beta1.reshape(1, _C)))
    p2 = _sc_scatter(y2.reshape(_N * _K, _C), g, dst)
    h2, s2, q2 = _stats_stage(p2.reshape(_NSC, _NPAD, _C))

    cam_pad = jnp.zeros((8, _CAM), jnp.float32).at[:4].set(camera_cond)
    bidx3 = batch_idx.reshape(_NB, 1, _R)
    return _final_stage(h2, s2, q2, gamma2.reshape(1, _C),
                        beta2.reshape(1, _C), cam_pad, Wc,
                        bc.reshape(1, 2 * _C), bidx3, x_features)


# speedup vs baseline: 3.8191x; 1.4757x over previous
import functools

import jax
import jax.numpy as jnp
from jax import lax
from jax.experimental import pallas as pl
from jax.experimental.pallas import tpu as pltpu
from jax.experimental.pallas import tpu_sc as plsc

_N = 10000
_E = 320000
_C = 128
_K = 27
_CAM = 256
_EPS = 1e-5

_NSC = 2
_NSUB = 16
_NT = _NSC * _NSUB
_EP = _E // _NT
_CH = 80
_NCH = _EP // _CH
_NPAD = 10240
_RPT = _NPAD // _NSUB
_RCH = 80
_NB = 25
_R = _N // _NB


def _dense_body(apply_bn, x_ref, w_ref, *rest):
    if apply_bn:
        s_ref, q_ref, g_ref, b_ref, y_ref = rest
        inv_n = jnp.float32(1.0 / _N)
        mu = s_ref[...] * inv_n
        var = q_ref[...] * inv_n - mu * mu
        hn = g_ref[...] * (x_ref[...] - mu) * lax.rsqrt(var + _EPS)
        xb = jnp.maximum(hn + b_ref[...], 0.0)
    else:
        (y_ref,) = rest
        xb = x_ref[...]
    y = jnp.dot(xb.astype(jnp.bfloat16), w_ref[...],
                preferred_element_type=jnp.float32)
    for k in range(_K):
        y_ref[k] = y[:, k * _C:(k + 1) * _C]


def _dense_stage(x, Wwide, stats=None):
    apply_bn = stats is not None
    in_specs = [
        pl.BlockSpec((_R, _C), lambda i: (i, 0)),
        pl.BlockSpec((_C, _K * _C), lambda i: (0, 0)),
    ]
    args = [x, Wwide]
    if apply_bn:
        in_specs += [pl.BlockSpec((1, _C), lambda i: (0, 0))] * 4
        args += list(stats)
    return pl.pallas_call(
        functools.partial(_dense_body, apply_bn),
        grid=(_NB,),
        in_specs=in_specs,
        out_specs=pl.BlockSpec((_K, _R, _C), lambda i: (0, i, 0)),
        out_shape=jax.ShapeDtypeStruct((_K, _N, _C), jnp.float32),
        compiler_params=pltpu.CompilerParams(
            dimension_semantics=("arbitrary",)),
    )(*args)


def _prep_body(src_ref, koff_ref, g_ref):
    g_ref[...] = koff_ref[...] * _N + src_ref[...]


def _prep_stage(src, koff):
    s2 = src.reshape(_E // 128, 128)
    k2 = koff.reshape(_E // 128, 128)
    g2 = pl.pallas_call(
        _prep_body,
        out_shape=jax.ShapeDtypeStruct((_E // 128, 128), jnp.int32),
    )(s2, k2)
    return g2.reshape(_E)


def _sc_scatter(table, g, dst):
    mesh = plsc.VectorSubcoreMesh(core_axis_name="c", subcore_axis_name="s")

    @functools.partial(
        pl.kernel,
        out_type=jax.ShapeDtypeStruct((_NSC * _NPAD, _C), jnp.float32),
        mesh=mesh,
        scratch_types=[
            pltpu.VMEM((_EP,), jnp.int32),
            pltpu.VMEM((_EP,), jnp.int32),
            pltpu.VMEM((_CH,), jnp.int32),
            pltpu.VMEM((_CH,), jnp.int32),
            pltpu.VMEM((_CH, _C), jnp.float32),
            pltpu.VMEM((_CH, _C), jnp.float32),
            pltpu.VMEM((8, _C), jnp.float32),
            pltpu.VMEM_SHARED((_NPAD, _C), jnp.float32),
            pltpu.SemaphoreType.DMA,
            pltpu.SemaphoreType.DMA,
        ],
    )
    def sc_kernel(table_h, g_h, dst_h, out_h,
                  g_v, d_v, db0_v, db1_v, rows0_v, rows1_v, zb_v, acc_s,
                  sem0, sem1):
        cid = lax.axis_index("c")
        sid = lax.axis_index("s")
        wid = sid * _NSC + cid
        ebase = pl.multiple_of(wid * _EP, 8)

        pltpu.sync_copy(g_h.at[pl.ds(ebase, _EP)], g_v)
        pltpu.sync_copy(dst_h.at[pl.ds(ebase, _EP)], d_v)

        def zvec(i, c):
            def zlane(j, c2):
                zb_v[i, pl.ds(j * 16, 16)] = jnp.zeros((16,), jnp.float32)
                return c2
            return lax.fori_loop(0, _C // 16, zlane, c)
        lax.fori_loop(0, 8, zvec, 0)

        rbase = sid * _RPT

        def zrow(j, c):
            pltpu.sync_copy(zb_v, acc_s.at[pl.ds(rbase + j * 8, 8)])
            return c
        lax.fori_loop(0, _RPT // 8, zrow, 0)
        plsc.subcore_barrier()

        def cp(eb, db):
            def cpb(j, c2):
                o = j * 16
                db[pl.ds(o, 16)] = d_v[pl.ds(eb + o, 16)]
                return c2
            lax.fori_loop(0, _CH // 16, cpb, 0)

        def pair(p, c):
            eb0 = pl.multiple_of(p * (2 * _CH), 8)
            eb1 = pl.multiple_of(p * (2 * _CH) + _CH, 8)
            cp(eb0, db0_v)
            cpy0 = pltpu.async_copy(table_h.at[g_v.at[pl.ds(eb0, _CH)]],
                                    rows0_v, sem0)
            cp(eb1, db1_v)
            cpy1 = pltpu.async_copy(table_h.at[g_v.at[pl.ds(eb1, _CH)]],
                                    rows1_v, sem1)
            cpy0.wait()
            pltpu.sync_copy(rows0_v, acc_s.at[db0_v], add=True)
            cpy1.wait()
            pltpu.sync_copy(rows1_v, acc_s.at[db1_v], add=True)
            return c
        lax.fori_loop(0, _NCH // 2, pair, 0)

        ebt = pl.multiple_of((_NCH - 1) * _CH, 8)
        cp(ebt, db0_v)
        pltpu.async_copy(table_h.at[g_v.at[pl.ds(ebt, _CH)]], rows0_v,
                         sem0).wait()
        pltpu.sync_copy(rows0_v, acc_s.at[db0_v], add=True)
        plsc.subcore_barrier()

        obase = cid * _NPAD + rbase

        def drain(j, c):
            pltpu.sync_copy(acc_s.at[pl.ds(rbase + j * 8, 8)], zb_v)
            pltpu.sync_copy(zb_v, out_h.at[pl.ds(obase + j * 8, 8)])
            return c
        lax.fori_loop(0, _RPT // 8, drain, 0)

    return sc_kernel(table, g, dst)


def _stats_body(p_ref, h_ref, sum_ref, sq_ref):
    i = pl.program_id(0)
    h = p_ref[0] + p_ref[1]
    h_ref[...] = h
    s = jnp.sum(h, axis=0, keepdims=True)
    q = jnp.sum(h * h, axis=0, keepdims=True)

    @pl.when(i == 0)
    def _():
        sum_ref[...] = s
        sq_ref[...] = q

    @pl.when(i > 0)
    def _():
        sum_ref[...] = sum_ref[...] + s
        sq_ref[...] = sq_ref[...] + q


def _stats_stage(partials):
    return pl.pallas_call(
        _stats_body,
        grid=(_NB,),
        in_specs=[pl.BlockSpec((_NSC, _R, _C), lambda i: (0, i, 0))],
        out_specs=[
            pl.BlockSpec((_R, _C), lambda i: (i, 0)),
            pl.BlockSpec((1, _C), lambda i: (0, 0)),
            pl.BlockSpec((1, _C), lambda i: (0, 0)),
        ],
        out_shape=[
            jax.ShapeDtypeStruct((_N, _C), jnp.float32),
            jax.ShapeDtypeStruct((1, _C), jnp.float32),
            jax.ShapeDtypeStruct((1, _C), jnp.float32),
        ],
        compiler_params=pltpu.CompilerParams(
            dimension_semantics=("arbitrary",)),
    )(partials)


def _final_body(h_ref, s_ref, q_ref, g_ref, b_ref, cam_ref, wc_ref, bc_ref,
                bidx_ref, x_ref, o_ref):
    inv_n = jnp.float32(1.0 / _N)
    mu = s_ref[...] * inv_n
    var = q_ref[...] * inv_n - mu * mu
    hn = g_ref[...] * (h_ref[...] - mu) * lax.rsqrt(var + _EPS) + b_ref[...]
    cam = jnp.dot(cam_ref[...], wc_ref[...],
                  preferred_element_type=jnp.float32) + bc_ref[...]
    bi = bidx_ref[0, 0, :]
    onehot = (bi[:, None] == lax.broadcasted_iota(jnp.int32, (1, 8), 1)
              ).astype(jnp.float32)
    film = jnp.dot(onehot, cam, preferred_element_type=jnp.float32)
    scale = film[:, :_C]
    shift = film[:, _C:]
    o_ref[...] = jnp.maximum(hn * (1.0 + scale) + shift, 0.0) + x_ref[...]


def _final_stage(h, s, q, gamma, beta, cam_pad, Wc, bc, bidx3, x):
    return pl.pallas_call(
        _final_body,
        grid=(_NB,),
        in_specs=[
            pl.BlockSpec((_R, _C), lambda i: (i, 0)),
            pl.BlockSpec((1, _C), lambda i: (0, 0)),
            pl.BlockSpec((1, _C), lambda i: (0, 0)),
            pl.BlockSpec((1, _C), lambda i: (0, 0)),
            pl.BlockSpec((1, _C), lambda i: (0, 0)),
            pl.BlockSpec((8, 2 * _C), lambda i: (0, 0)),
            pl.BlockSpec((_CAM, 2 * _C), lambda i: (0, 0)),
            pl.BlockSpec((1, 2 * _C), lambda i: (0, 0)),
            pl.BlockSpec((1, 1, _R), lambda i: (i, 0, 0)),
            pl.BlockSpec((_R, _C), lambda i: (i, 0)),
        ],
        out_specs=pl.BlockSpec((_R, _C), lambda i: (i, 0)),
        out_shape=jax.ShapeDtypeStruct((_N, _C), jnp.float32),
        compiler_params=pltpu.CompilerParams(
            dimension_semantics=("arbitrary",)),
    )(h, s, q, gamma, beta, cam_pad, Wc, bc, bidx3, x)


def kernel(x_features, camera_cond, W1, b1, gamma1, beta1, W2, b2, gamma2,
           beta2, Wc, bc, edge_index, kernel_offsets, batch_idx):
    del b1, b2
    src = edge_index[0]
    dst = edge_index[1]
    g = _prep_stage(src, kernel_offsets)
    W1w = W1.transpose(1, 0, 2).reshape(_C, _K * _C).astype(jnp.bfloat16)
    W2w = W2.transpose(1, 0, 2).reshape(_C, _K * _C).astype(jnp.bfloat16)

    y1 = _dense_stage(x_features, W1w)
    p1 = _sc_scatter(y1.reshape(_K * _N, _C), g, dst)
    h1, s1, q1 = _stats_stage(p1.reshape(_NSC, _NPAD, _C))

    y2 = _dense_stage(h1, W2w, stats=(s1, q1, gamma1.reshape(1, _C),
                                      beta1.reshape(1, _C)))
    p2 = _sc_scatter(y2.reshape(_K * _N, _C), g, dst)
    h2, s2, q2 = _stats_stage(p2.reshape(_NSC, _NPAD, _C))

    cam_pad = jnp.zeros((8, _CAM), jnp.float32).at[:4].set(camera_cond)
    bidx3 = batch_idx.reshape(_NB, 1, _R)
    return _final_stage(h2, s2, q2, gamma2.reshape(1, _C),
                        beta2.reshape(1, _C), cam_pad, Wc,
                        bc.reshape(1, 2 * _C), bidx3, x_features)


# speedup vs baseline: 3.8704x; 1.0134x over previous
import functools

import jax
import jax.numpy as jnp
from jax import lax
from jax.experimental import pallas as pl
from jax.experimental.pallas import tpu as pltpu
from jax.experimental.pallas import tpu_sc as plsc

_N = 10000
_E = 320000
_C = 128
_K = 27
_CAM = 256
_EPS = 1e-5

_NSC = 2
_NSUB = 16
_NT = _NSC * _NSUB
_EP = _E // _NT
_CH = 80
_NCH = _EP // _CH
_NPAD = 10240
_RPT = _NPAD // _NSUB
_RCH = 80
_NB = 25
_R = _N // _NB


def _dense_body(apply_bn, x_ref, w_ref, *rest):
    if apply_bn:
        s_ref, q_ref, g_ref, b_ref, y_ref = rest
        inv_n = jnp.float32(1.0 / _N)
        mu = s_ref[...] * inv_n
        var = q_ref[...] * inv_n - mu * mu
        hn = g_ref[...] * (x_ref[...] - mu) * lax.rsqrt(var + _EPS)
        xb = jnp.maximum(hn + b_ref[...], 0.0)
    else:
        (y_ref,) = rest
        xb = x_ref[...]
    y = jnp.dot(xb.astype(jnp.bfloat16), w_ref[...],
                preferred_element_type=jnp.float32)
    for k in range(_K):
        y_ref[k] = y[:, k * _C:(k + 1) * _C]


def _dense_stage(x, Wwide, stats=None):
    apply_bn = stats is not None
    in_specs = [
        pl.BlockSpec((_R, _C), lambda i: (i, 0)),
        pl.BlockSpec((_C, _K * _C), lambda i: (0, 0)),
    ]
    args = [x, Wwide]
    if apply_bn:
        in_specs += [pl.BlockSpec((1, _C), lambda i: (0, 0))] * 4
        args += list(stats)
    return pl.pallas_call(
        functools.partial(_dense_body, apply_bn),
        grid=(_NB,),
        in_specs=in_specs,
        out_specs=pl.BlockSpec((_K, _R, _C), lambda i: (0, i, 0)),
        out_shape=jax.ShapeDtypeStruct((_K, _N, _C), jnp.float32),
        compiler_params=pltpu.CompilerParams(
            dimension_semantics=("arbitrary",)),
    )(*args)


def _prep_body(src_ref, koff_ref, g_ref):
    g_ref[...] = koff_ref[...] * _N + src_ref[...]


def _prep_stage(src, koff):
    s2 = src.reshape(_E // 128, 128)
    k2 = koff.reshape(_E // 128, 128)
    g2 = pl.pallas_call(
        _prep_body,
        out_shape=jax.ShapeDtypeStruct((_E // 128, 128), jnp.int32),
    )(s2, k2)
    return g2.reshape(_E)


def _sc_scatter(table, g, dst):
    mesh = plsc.VectorSubcoreMesh(core_axis_name="c", subcore_axis_name="s")

    @functools.partial(
        pl.kernel,
        out_type=jax.ShapeDtypeStruct((_NSC * _NPAD, _C), jnp.float32),
        mesh=mesh,
        scratch_types=[
            pltpu.VMEM((_EP,), jnp.int32),
            pltpu.VMEM((_EP,), jnp.int32),
            pltpu.VMEM((_CH,), jnp.int32),
            pltpu.VMEM((_CH,), jnp.int32),
            pltpu.VMEM((_CH, _C), jnp.float32),
            pltpu.VMEM((_CH, _C), jnp.float32),
            pltpu.VMEM((8, _C), jnp.float32),
            pltpu.VMEM_SHARED((_NPAD, _C), jnp.float32),
            pltpu.SemaphoreType.DMA,
            pltpu.SemaphoreType.DMA,
            pltpu.SemaphoreType.DMA,
            pltpu.SemaphoreType.DMA,
        ],
    )
    def sc_kernel(table_h, g_h, dst_h, out_h,
                  g_v, d_v, db0_v, db1_v, rows0_v, rows1_v, zb_v, acc_s,
                  sem0, sem1, ssem0, ssem1):
        cid = lax.axis_index("c")
        sid = lax.axis_index("s")
        wid = sid * _NSC + cid
        ebase = pl.multiple_of(wid * _EP, 8)

        pltpu.sync_copy(g_h.at[pl.ds(ebase, _EP)], g_v)
        pltpu.sync_copy(dst_h.at[pl.ds(ebase, _EP)], d_v)

        def zvec(i, c):
            def zlane(j, c2):
                zb_v[i, pl.ds(j * 16, 16)] = jnp.zeros((16,), jnp.float32)
                return c2
            return lax.fori_loop(0, _C // 16, zlane, c)
        lax.fori_loop(0, 8, zvec, 0)

        rbase = sid * _RPT

        def zrow(j, c):
            pltpu.sync_copy(zb_v, acc_s.at[pl.ds(rbase + j * 8, 8)])
            return c
        lax.fori_loop(0, _RPT // 8, zrow, 0)
        plsc.subcore_barrier()

        def cp(eb, db):
            def cpb(j, c2):
                o = j * 16
                db[pl.ds(o, 16)] = d_v[pl.ds(eb + o, 16)]
                return c2
            lax.fori_loop(0, _CH // 16, cpb, 0)

        def pair(p, c):
            eb0 = pl.multiple_of(p * (2 * _CH), 8)
            eb1 = pl.multiple_of(p * (2 * _CH) + _CH, 8)
            cp(eb0, db0_v)
            cpy0 = pltpu.async_copy(table_h.at[g_v.at[pl.ds(eb0, _CH)]],
                                    rows0_v, sem0)
            cp(eb1, db1_v)
            cpy1 = pltpu.async_copy(table_h.at[g_v.at[pl.ds(eb1, _CH)]],
                                    rows1_v, sem1)
            cpy0.wait()
            sc0 = pltpu.async_copy(rows0_v, acc_s.at[db0_v], ssem0, add=True)
            cpy1.wait()
            sc1 = pltpu.async_copy(rows1_v, acc_s.at[db1_v], ssem1, add=True)
            sc0.wait()
            sc1.wait()
            return c
        lax.fori_loop(0, _NCH // 2, pair, 0)

        ebt = pl.multiple_of((_NCH - 1) * _CH, 8)
        cp(ebt, db0_v)
        pltpu.async_copy(table_h.at[g_v.at[pl.ds(ebt, _CH)]], rows0_v,
                         sem0).wait()
        pltpu.sync_copy(rows0_v, acc_s.at[db0_v], add=True)
        plsc.subcore_barrier()

        obase = cid * _NPAD + rbase

        def drain(j, c):
            pltpu.sync_copy(acc_s.at[pl.ds(rbase + j * 8, 8)], zb_v)
            pltpu.sync_copy(zb_v, out_h.at[pl.ds(obase + j * 8, 8)])
            return c
        lax.fori_loop(0, _RPT // 8, drain, 0)

    return sc_kernel(table, g, dst)


def _stats_body(p_ref, h_ref, sum_ref, sq_ref):
    i = pl.program_id(0)
    h = p_ref[0] + p_ref[1]
    h_ref[...] = h
    s = jnp.sum(h, axis=0, keepdims=True)
    q = jnp.sum(h * h, axis=0, keepdims=True)

    @pl.when(i == 0)
    def _():
        sum_ref[...] = s
        sq_ref[...] = q

    @pl.when(i > 0)
    def _():
        sum_ref[...] = sum_ref[...] + s
        sq_ref[...] = sq_ref[...] + q


def _stats_stage(partials):
    return pl.pallas_call(
        _stats_body,
        grid=(_NB,),
        in_specs=[pl.BlockSpec((_NSC, _R, _C), lambda i: (0, i, 0))],
        out_specs=[
            pl.BlockSpec((_R, _C), lambda i: (i, 0)),
            pl.BlockSpec((1, _C), lambda i: (0, 0)),
            pl.BlockSpec((1, _C), lambda i: (0, 0)),
        ],
        out_shape=[
            jax.ShapeDtypeStruct((_N, _C), jnp.float32),
            jax.ShapeDtypeStruct((1, _C), jnp.float32),
            jax.ShapeDtypeStruct((1, _C), jnp.float32),
        ],
        compiler_params=pltpu.CompilerParams(
            dimension_semantics=("arbitrary",)),
    )(partials)


def _final_body(h_ref, s_ref, q_ref, g_ref, b_ref, cam_ref, wc_ref, bc_ref,
                bidx_ref, x_ref, o_ref):
    inv_n = jnp.float32(1.0 / _N)
    mu = s_ref[...] * inv_n
    var = q_ref[...] * inv_n - mu * mu
    hn = g_ref[...] * (h_ref[...] - mu) * lax.rsqrt(var + _EPS) + b_ref[...]
    cam = jnp.dot(cam_ref[...], wc_ref[...],
                  preferred_element_type=jnp.float32) + bc_ref[...]
    bi = bidx_ref[0, 0, :]
    onehot = (bi[:, None] == lax.broadcasted_iota(jnp.int32, (1, 8), 1)
              ).astype(jnp.float32)
    film = jnp.dot(onehot, cam, preferred_element_type=jnp.float32)
    scale = film[:, :_C]
    shift = film[:, _C:]
    o_ref[...] = jnp.maximum(hn * (1.0 + scale) + shift, 0.0) + x_ref[...]


def _final_stage(h, s, q, gamma, beta, cam_pad, Wc, bc, bidx3, x):
    return pl.pallas_call(
        _final_body,
        grid=(_NB,),
        in_specs=[
            pl.BlockSpec((_R, _C), lambda i: (i, 0)),
            pl.BlockSpec((1, _C), lambda i: (0, 0)),
            pl.BlockSpec((1, _C), lambda i: (0, 0)),
            pl.BlockSpec((1, _C), lambda i: (0, 0)),
            pl.BlockSpec((1, _C), lambda i: (0, 0)),
            pl.BlockSpec((8, 2 * _C), lambda i: (0, 0)),
            pl.BlockSpec((_CAM, 2 * _C), lambda i: (0, 0)),
            pl.BlockSpec((1, 2 * _C), lambda i: (0, 0)),
            pl.BlockSpec((1, 1, _R), lambda i: (i, 0, 0)),
            pl.BlockSpec((_R, _C), lambda i: (i, 0)),
        ],
        out_specs=pl.BlockSpec((_R, _C), lambda i: (i, 0)),
        out_shape=jax.ShapeDtypeStruct((_N, _C), jnp.float32),
        compiler_params=pltpu.CompilerParams(
            dimension_semantics=("arbitrary",)),
    )(h, s, q, gamma, beta, cam_pad, Wc, bc, bidx3, x)


def kernel(x_features, camera_cond, W1, b1, gamma1, beta1, W2, b2, gamma2,
           beta2, Wc, bc, edge_index, kernel_offsets, batch_idx):
    del b1, b2
    src = edge_index[0]
    dst = edge_index[1]
    g = _prep_stage(src, kernel_offsets)
    W1w = W1.transpose(1, 0, 2).reshape(_C, _K * _C).astype(jnp.bfloat16)
    W2w = W2.transpose(1, 0, 2).reshape(_C, _K * _C).astype(jnp.bfloat16)

    y1 = _dense_stage(x_features, W1w)
    p1 = _sc_scatter(y1.reshape(_K * _N, _C), g, dst)
    h1, s1, q1 = _stats_stage(p1.reshape(_NSC, _NPAD, _C))

    y2 = _dense_stage(h1, W2w, stats=(s1, q1, gamma1.reshape(1, _C),
                                      beta1.reshape(1, _C)))
    p2 = _sc_scatter(y2.reshape(_K * _N, _C), g, dst)
    h2, s2, q2 = _stats_stage(p2.reshape(_NSC, _NPAD, _C))

    cam_pad = jnp.zeros((8, _CAM), jnp.float32).at[:4].set(camera_cond)
    bidx3 = batch_idx.reshape(_NB, 1, _R)
    return _final_stage(h2, s2, q2, gamma2.reshape(1, _C),
                        beta2.reshape(1, _C), cam_pad, Wc,
                        bc.reshape(1, 2 * _C), bidx3, x_features)


# speedup vs baseline: 3.9190x; 1.0126x over previous
import functools

import jax
import jax.numpy as jnp
from jax import lax
from jax.experimental import pallas as pl
from jax.experimental.pallas import tpu as pltpu
from jax.experimental.pallas import tpu_sc as plsc

_N = 10000
_E = 320000
_C = 128
_K = 27
_CAM = 256
_EPS = 1e-5

_NSC = 2
_NSUB = 16
_NT = _NSC * _NSUB
_EP = _E // _NT
_CH = 80
_NCH = _EP // _CH
_NPAD = 10240
_RPT = _NPAD // _NSUB
_RCH = 80
_NB = 25
_R = _N // _NB


def _dense_body(apply_bn, x_ref, w_ref, *rest):
    if apply_bn:
        s_ref, q_ref, g_ref, b_ref, y_ref = rest
        inv_n = jnp.float32(1.0 / _N)
        mu = s_ref[...] * inv_n
        var = q_ref[...] * inv_n - mu * mu
        hn = g_ref[...] * (x_ref[...] - mu) * lax.rsqrt(var + _EPS)
        xb = jnp.maximum(hn + b_ref[...], 0.0)
    else:
        (y_ref,) = rest
        xb = x_ref[...]
    y = jnp.dot(xb.astype(jnp.bfloat16), w_ref[...],
                preferred_element_type=jnp.float32)
    for k in range(_K):
        y_ref[k] = y[:, k * _C:(k + 1) * _C]


def _dense_stage(x, Wwide, stats=None):
    apply_bn = stats is not None
    in_specs = [
        pl.BlockSpec((_R, _C), lambda i: (i, 0)),
        pl.BlockSpec((_C, _K * _C), lambda i: (0, 0)),
    ]
    args = [x, Wwide]
    if apply_bn:
        in_specs += [pl.BlockSpec((1, _C), lambda i: (0, 0))] * 4
        args += list(stats)
    return pl.pallas_call(
        functools.partial(_dense_body, apply_bn),
        grid=(_NB,),
        in_specs=in_specs,
        out_specs=pl.BlockSpec((_K, _R, _C), lambda i: (0, i, 0)),
        out_shape=jax.ShapeDtypeStruct((_K, _N, _C), jnp.float32),
        compiler_params=pltpu.CompilerParams(
            dimension_semantics=("arbitrary",)),
    )(*args)


def _prep_body(src_ref, koff_ref, g_ref):
    g_ref[...] = koff_ref[...] * _N + src_ref[...]


def _prep_stage(src, koff):
    s2 = src.reshape(_E // 128, 128)
    k2 = koff.reshape(_E // 128, 128)
    g2 = pl.pallas_call(
        _prep_body,
        out_shape=jax.ShapeDtypeStruct((_E // 128, 128), jnp.int32),
    )(s2, k2)
    return g2.reshape(_E)


def _sc_scatter(table, g, dst):
    mesh = plsc.VectorSubcoreMesh(core_axis_name="c", subcore_axis_name="s")

    @functools.partial(
        pl.kernel,
        out_type=jax.ShapeDtypeStruct((_NSC * _NPAD, _C), jnp.float32),
        mesh=mesh,
        scratch_types=[
            pltpu.VMEM((_EP,), jnp.int32),
            pltpu.VMEM((_EP,), jnp.int32),
            pltpu.VMEM((_CH,), jnp.int32),
            pltpu.VMEM((_CH,), jnp.int32),
            pltpu.VMEM((_CH, _C), jnp.float32),
            pltpu.VMEM((_CH, _C), jnp.float32),
            pltpu.VMEM((8, _C), jnp.float32),
            pltpu.VMEM_SHARED((_NPAD, _C), jnp.float32),
            pltpu.SemaphoreType.DMA,
            pltpu.SemaphoreType.DMA,
            pltpu.SemaphoreType.DMA,
            pltpu.SemaphoreType.DMA,
        ],
    )
    def sc_kernel(table_h, g_h, dst_h, out_h,
                  g_v, d_v, db0_v, db1_v, rows0_v, rows1_v, zb_v, acc_s,
                  sem0, sem1, ssem0, ssem1):
        cid = lax.axis_index("c")
        sid = lax.axis_index("s")
        wid = sid * _NSC + cid
        ebase = pl.multiple_of(wid * _EP, 8)

        pltpu.sync_copy(g_h.at[pl.ds(ebase, _EP)], g_v)
        pltpu.sync_copy(dst_h.at[pl.ds(ebase, _EP)], d_v)

        def zvec(i, c):
            def zlane(j, c2):
                zb_v[i, pl.ds(j * 16, 16)] = jnp.zeros((16,), jnp.float32)
                return c2
            return lax.fori_loop(0, _C // 16, zlane, c)
        lax.fori_loop(0, 8, zvec, 0)

        rbase = sid * _RPT

        def zrow(j, c):
            pltpu.sync_copy(zb_v, acc_s.at[pl.ds(rbase + j * 8, 8)])
            return c
        lax.fori_loop(0, _RPT // 8, zrow, 0)
        plsc.subcore_barrier()

        def cp(eb, db):
            def cpb(j, c2):
                o = j * 16
                db[pl.ds(o, 16)] = d_v[pl.ds(eb + o, 16)]
                return c2
            lax.fori_loop(0, _CH // 16, cpb, 0)

        def gather(i, rows, sem):
            eb = pl.multiple_of(i * _CH, 8)
            return pltpu.async_copy(table_h.at[g_v.at[pl.ds(eb, _CH)]],
                                    rows, sem)

        def gather_wait(i, rows, sem):
            eb = pl.multiple_of(i * _CH, 8)
            pltpu.make_async_copy(table_h.at[g_v.at[pl.ds(eb, _CH)]],
                                  rows, sem).wait()

        def scatter(rows, db, sem):
            return pltpu.async_copy(rows, acc_s.at[db], sem, add=True)

        def scatter_wait(rows, db, sem):
            pltpu.make_async_copy(rows, acc_s.at[db], sem).wait()

        cp(0, db0_v)
        gather(0, rows0_v, sem0)
        cp(_CH, db1_v)
        gather(1, rows1_v, sem1)

        def round_(r, c):
            i0 = 2 * r
            gather_wait(i0, rows0_v, sem0)
            scatter(rows0_v, db0_v, ssem0)
            gather_wait(i0 + 1, rows1_v, sem1)
            scatter(rows1_v, db1_v, ssem1)
            scatter_wait(rows0_v, db0_v, ssem0)
            cp((i0 + 2) * _CH, db0_v)
            gather(i0 + 2, rows0_v, sem0)
            scatter_wait(rows1_v, db1_v, ssem1)
            cp((i0 + 3) * _CH, db1_v)
            gather(i0 + 3, rows1_v, sem1)
            return c
        lax.fori_loop(0, (_NCH - 3) // 2, round_, 0)

        gather_wait(_NCH - 3, rows0_v, sem0)
        scatter(rows0_v, db0_v, ssem0)
        gather_wait(_NCH - 2, rows1_v, sem1)
        scatter(rows1_v, db1_v, ssem1)
        scatter_wait(rows0_v, db0_v, ssem0)
        cp((_NCH - 1) * _CH, db0_v)
        gather(_NCH - 1, rows0_v, sem0).wait()
        pltpu.sync_copy(rows0_v, acc_s.at[db0_v], add=True)
        scatter_wait(rows1_v, db1_v, ssem1)
        plsc.subcore_barrier()

        obase = cid * _NPAD + rbase

        def drain(j, c):
            pltpu.sync_copy(acc_s.at[pl.ds(rbase + j * 8, 8)], zb_v)
            pltpu.sync_copy(zb_v, out_h.at[pl.ds(obase + j * 8, 8)])
            return c
        lax.fori_loop(0, _RPT // 8, drain, 0)

    return sc_kernel(table, g, dst)


def _stats_body(p_ref, h_ref, sum_ref, sq_ref):
    i = pl.program_id(0)
    h = p_ref[0] + p_ref[1]
    h_ref[...] = h
    s = jnp.sum(h, axis=0, keepdims=True)
    q = jnp.sum(h * h, axis=0, keepdims=True)

    @pl.when(i == 0)
    def _():
        sum_ref[...] = s
        sq_ref[...] = q

    @pl.when(i > 0)
    def _():
        sum_ref[...] = sum_ref[...] + s
        sq_ref[...] = sq_ref[...] + q


def _stats_stage(partials):
    return pl.pallas_call(
        _stats_body,
        grid=(_NB,),
        in_specs=[pl.BlockSpec((_NSC, _R, _C), lambda i: (0, i, 0))],
        out_specs=[
            pl.BlockSpec((_R, _C), lambda i: (i, 0)),
            pl.BlockSpec((1, _C), lambda i: (0, 0)),
            pl.BlockSpec((1, _C), lambda i: (0, 0)),
        ],
        out_shape=[
            jax.ShapeDtypeStruct((_N, _C), jnp.float32),
            jax.ShapeDtypeStruct((1, _C), jnp.float32),
            jax.ShapeDtypeStruct((1, _C), jnp.float32),
        ],
        compiler_params=pltpu.CompilerParams(
            dimension_semantics=("arbitrary",)),
    )(partials)


def _final_body(h_ref, s_ref, q_ref, g_ref, b_ref, cam_ref, wc_ref, bc_ref,
                bidx_ref, x_ref, o_ref):
    inv_n = jnp.float32(1.0 / _N)
    mu = s_ref[...] * inv_n
    var = q_ref[...] * inv_n - mu * mu
    hn = g_ref[...] * (h_ref[...] - mu) * lax.rsqrt(var + _EPS) + b_ref[...]
    cam = jnp.dot(cam_ref[...], wc_ref[...],
                  preferred_element_type=jnp.float32) + bc_ref[...]
    bi = bidx_ref[0, 0, :]
    onehot = (bi[:, None] == lax.broadcasted_iota(jnp.int32, (1, 8), 1)
              ).astype(jnp.float32)
    film = jnp.dot(onehot, cam, preferred_element_type=jnp.float32)
    scale = film[:, :_C]
    shift = film[:, _C:]
    o_ref[...] = jnp.maximum(hn * (1.0 + scale) + shift, 0.0) + x_ref[...]


def _final_stage(h, s, q, gamma, beta, cam_pad, Wc, bc, bidx3, x):
    return pl.pallas_call(
        _final_body,
        grid=(_NB,),
        in_specs=[
            pl.BlockSpec((_R, _C), lambda i: (i, 0)),
            pl.BlockSpec((1, _C), lambda i: (0, 0)),
            pl.BlockSpec((1, _C), lambda i: (0, 0)),
            pl.BlockSpec((1, _C), lambda i: (0, 0)),
            pl.BlockSpec((1, _C), lambda i: (0, 0)),
            pl.BlockSpec((8, 2 * _C), lambda i: (0, 0)),
            pl.BlockSpec((_CAM, 2 * _C), lambda i: (0, 0)),
            pl.BlockSpec((1, 2 * _C), lambda i: (0, 0)),
            pl.BlockSpec((1, 1, _R), lambda i: (i, 0, 0)),
            pl.BlockSpec((_R, _C), lambda i: (i, 0)),
        ],
        out_specs=pl.BlockSpec((_R, _C), lambda i: (i, 0)),
        out_shape=jax.ShapeDtypeStruct((_N, _C), jnp.float32),
        compiler_params=pltpu.CompilerParams(
            dimension_semantics=("arbitrary",)),
    )(h, s, q, gamma, beta, cam_pad, Wc, bc, bidx3, x)


def kernel(x_features, camera_cond, W1, b1, gamma1, beta1, W2, b2, gamma2,
           beta2, Wc, bc, edge_index, kernel_offsets, batch_idx):
    del b1, b2
    src = edge_index[0]
    dst = edge_index[1]
    g = _prep_stage(src, kernel_offsets)
    W1w = W1.transpose(1, 0, 2).reshape(_C, _K * _C).astype(jnp.bfloat16)
    W2w = W2.transpose(1, 0, 2).reshape(_C, _K * _C).astype(jnp.bfloat16)

    y1 = _dense_stage(x_features, W1w)
    p1 = _sc_scatter(y1.reshape(_K * _N, _C), g, dst)
    h1, s1, q1 = _stats_stage(p1.reshape(_NSC, _NPAD, _C))

    y2 = _dense_stage(h1, W2w, stats=(s1, q1, gamma1.reshape(1, _C),
                                      beta1.reshape(1, _C)))
    p2 = _sc_scatter(y2.reshape(_K * _N, _C), g, dst)
    h2, s2, q2 = _stats_stage(p2.reshape(_NSC, _NPAD, _C))

    cam_pad = jnp.zeros((8, _CAM), jnp.float32).at[:4].set(camera_cond)
    bidx3 = batch_idx.reshape(_NB, 1, _R)
    return _final_stage(h2, s2, q2, gamma2.reshape(1, _C),
                        beta2.reshape(1, _C), cam_pad, Wc,
                        bc.reshape(1, 2 * _C), bidx3, x_features)


# speedup vs baseline: 4.0939x; 1.0446x over previous
import functools

import jax
import jax.numpy as jnp
from jax import lax
from jax.experimental import pallas as pl
from jax.experimental.pallas import tpu as pltpu
from jax.experimental.pallas import tpu_sc as plsc

_N = 10000
_E = 320000
_C = 128
_K = 27
_CAM = 256
_EPS = 1e-5

_NSC = 2
_NSUB = 16
_NT = _NSC * _NSUB
_EP = _E // _NT
_CH = 128
_NCH = _EP // _CH
_CHT = _EP - _NCH * _CH
_NPAD = 10240
_RPT = _NPAD // _NSUB
_NB = 25
_R = _N // _NB


def _dense_body(apply_bn, x_ref, w_ref, *rest):
    if apply_bn:
        s_ref, q_ref, g_ref, b_ref, y_ref = rest
        inv_n = jnp.float32(1.0 / _N)
        mu = s_ref[...] * inv_n
        var = q_ref[...] * inv_n - mu * mu
        hn = g_ref[...] * (x_ref[...] - mu) * lax.rsqrt(var + _EPS)
        xb = jnp.maximum(hn + b_ref[...], 0.0)
    else:
        (y_ref,) = rest
        xb = x_ref[...]
    y = jnp.dot(xb.astype(jnp.bfloat16), w_ref[...],
                preferred_element_type=jnp.float32)
    for k in range(_K):
        y_ref[k] = y[:, k * _C:(k + 1) * _C]


def _dense_stage(x, Wwide, stats=None):
    apply_bn = stats is not None
    in_specs = [
        pl.BlockSpec((_R, _C), lambda i: (i, 0)),
        pl.BlockSpec((_C, _K * _C), lambda i: (0, 0)),
    ]
    args = [x, Wwide]
    if apply_bn:
        in_specs += [pl.BlockSpec((1, _C), lambda i: (0, 0))] * 4
        args += list(stats)
    return pl.pallas_call(
        functools.partial(_dense_body, apply_bn),
        grid=(_NB,),
        in_specs=in_specs,
        out_specs=pl.BlockSpec((_K, _R, _C), lambda i: (0, i, 0)),
        out_shape=jax.ShapeDtypeStruct((_K, _N, _C), jnp.float32),
        compiler_params=pltpu.CompilerParams(
            dimension_semantics=("arbitrary",)),
    )(*args)


def _prep_body(src_ref, koff_ref, g_ref):
    g_ref[...] = koff_ref[...] * _N + src_ref[...]


def _prep_stage(src, koff):
    s2 = src.reshape(_E // 128, 128)
    k2 = koff.reshape(_E // 128, 128)
    g2 = pl.pallas_call(
        _prep_body,
        out_shape=jax.ShapeDtypeStruct((_E // 128, 128), jnp.int32),
    )(s2, k2)
    return g2.reshape(_E)


def _sc_scatter(table, g, dst, zrows):
    mesh = plsc.VectorSubcoreMesh(core_axis_name="c", subcore_axis_name="s")

    @functools.partial(
        pl.kernel,
        out_type=jax.ShapeDtypeStruct((_NSC * _NPAD, _C), jnp.float32),
        mesh=mesh,
        scratch_types=[
            pltpu.VMEM((_EP,), jnp.int32),
            pltpu.VMEM((_CH,), jnp.int32),
            pltpu.VMEM((_CH,), jnp.int32),
            pltpu.VMEM((_CHT,), jnp.int32),
            pltpu.VMEM((_CH, _C), jnp.float32),
            pltpu.VMEM((_CH, _C), jnp.float32),
            pltpu.VMEM((_CHT, _C), jnp.float32),
            pltpu.VMEM((8, _C), jnp.float32),
            pltpu.VMEM_SHARED((_NPAD, _C), jnp.float32),
            pltpu.SemaphoreType.DMA,
            pltpu.SemaphoreType.DMA,
            pltpu.SemaphoreType.DMA,
            pltpu.SemaphoreType.DMA,
            pltpu.SemaphoreType.DMA,
            pltpu.SemaphoreType.DMA,
        ],
    )
    def sc_kernel(table_h, g_h, dst_h, zrows_h, out_h,
                  g_v, db0_v, db1_v, dbt_v, rows0_v, rows1_v, rowt_v, zb_v,
                  acc_s, sem0, sem1, ssem0, ssem1, isem0, isem1):
        cid = lax.axis_index("c")
        sid = lax.axis_index("s")
        wid = sid * _NSC + cid
        ebase = pl.multiple_of(wid * _EP, 8)

        pltpu.sync_copy(g_h.at[pl.ds(ebase, _EP)], g_v)
        pltpu.sync_copy(zrows_h, zb_v)

        rbase = sid * _RPT

        def zrow(j, c):
            ro = pl.multiple_of(rbase + j * 8, 8)
            pltpu.sync_copy(zb_v, acc_s.at[pl.ds(ro, 8)])
            return c
        lax.fori_loop(0, _RPT // 8, zrow, 0)
        plsc.subcore_barrier()

        def idx_load(i, db, sem):
            eo = pl.multiple_of(ebase + i * _CH, 8)
            return pltpu.async_copy(dst_h.at[pl.ds(eo, _CH)], db, sem)

        def idx_wait(i, db, sem):
            eo = pl.multiple_of(ebase + i * _CH, 8)
            pltpu.make_async_copy(dst_h.at[pl.ds(eo, _CH)], db, sem).wait()

        def gather(i, rows, sem):
            eb = pl.multiple_of(i * _CH, 8)
            return pltpu.async_copy(table_h.at[g_v.at[pl.ds(eb, _CH)]],
                                    rows, sem)

        def gather_wait(i, rows, sem):
            eb = pl.multiple_of(i * _CH, 8)
            pltpu.make_async_copy(table_h.at[g_v.at[pl.ds(eb, _CH)]],
                                  rows, sem).wait()

        def scatter(rows, db, sem):
            return pltpu.async_copy(rows, acc_s.at[db], sem, add=True)

        def scatter_wait(rows, db, sem):
            pltpu.make_async_copy(rows, acc_s.at[db], sem).wait()

        idx_load(0, db0_v, isem0)
        gather(0, rows0_v, sem0)
        idx_load(1, db1_v, isem1)
        gather(1, rows1_v, sem1)

        def round_(r, c):
            i0 = 2 * r
            gather_wait(i0, rows0_v, sem0)
            idx_wait(i0, db0_v, isem0)
            scatter(rows0_v, db0_v, ssem0)
            gather_wait(i0 + 1, rows1_v, sem1)
            idx_wait(i0 + 1, db1_v, isem1)
            scatter(rows1_v, db1_v, ssem1)
            scatter_wait(rows0_v, db0_v, ssem0)
            idx_load(i0 + 2, db0_v, isem0)
            gather(i0 + 2, rows0_v, sem0)
            scatter_wait(rows1_v, db1_v, ssem1)
            idx_load(i0 + 3, db1_v, isem1)
            gather(i0 + 3, rows1_v, sem1)
            return c
        lax.fori_loop(0, _NCH // 2 - 1, round_, 0)

        gather_wait(_NCH - 2, rows0_v, sem0)
        idx_wait(_NCH - 2, db0_v, isem0)
        scatter(rows0_v, db0_v, ssem0)
        gather_wait(_NCH - 1, rows1_v, sem1)
        idx_wait(_NCH - 1, db1_v, isem1)
        scatter(rows1_v, db1_v, ssem1)
        ebt = pl.multiple_of(_NCH * _CH, 8)
        pltpu.sync_copy(dst_h.at[pl.ds(ebase + ebt, _CHT)], dbt_v)
        pltpu.async_copy(table_h.at[g_v.at[pl.ds(ebt, _CHT)]], rowt_v,
                         sem0).wait()
        scatter_wait(rows0_v, db0_v, ssem0)
        scatter_wait(rows1_v, db1_v, ssem1)
        pltpu.sync_copy(rowt_v, acc_s.at[dbt_v], add=True)
        plsc.subcore_barrier()

        obase = cid * _NPAD + rbase

        def drain(j, c):
            ro = pl.multiple_of(rbase + j * 8, 8)
            oo = pl.multiple_of(obase + j * 8, 8)
            pltpu.sync_copy(acc_s.at[pl.ds(ro, 8)], zb_v)
            pltpu.sync_copy(zb_v, out_h.at[pl.ds(oo, 8)])
            return c
        lax.fori_loop(0, _RPT // 8, drain, 0)

    return sc_kernel(table, g, dst, zrows)


def _stats_body(p_ref, h_ref, sum_ref, sq_ref):
    i = pl.program_id(0)
    h = p_ref[0] + p_ref[1]
    h_ref[...] = h
    s = jnp.sum(h, axis=0, keepdims=True)
    q = jnp.sum(h * h, axis=0, keepdims=True)

    @pl.when(i == 0)
    def _():
        sum_ref[...] = s
        sq_ref[...] = q

    @pl.when(i > 0)
    def _():
        sum_ref[...] = sum_ref[...] + s
        sq_ref[...] = sq_ref[...] + q


def _stats_stage(partials):
    return pl.pallas_call(
        _stats_body,
        grid=(_NB,),
        in_specs=[pl.BlockSpec((_NSC, _R, _C), lambda i: (0, i, 0))],
        out_specs=[
            pl.BlockSpec((_R, _C), lambda i: (i, 0)),
            pl.BlockSpec((1, _C), lambda i: (0, 0)),
            pl.BlockSpec((1, _C), lambda i: (0, 0)),
        ],
        out_shape=[
            jax.ShapeDtypeStruct((_N, _C), jnp.float32),
            jax.ShapeDtypeStruct((1, _C), jnp.float32),
            jax.ShapeDtypeStruct((1, _C), jnp.float32),
        ],
        compiler_params=pltpu.CompilerParams(
            dimension_semantics=("arbitrary",)),
    )(partials)


def _final_body(h_ref, s_ref, q_ref, g_ref, b_ref, cam_ref, wc_ref, bc_ref,
                bidx_ref, x_ref, o_ref):
    inv_n = jnp.float32(1.0 / _N)
    mu = s_ref[...] * inv_n
    var = q_ref[...] * inv_n - mu * mu
    hn = g_ref[...] * (h_ref[...] - mu) * lax.rsqrt(var + _EPS) + b_ref[...]
    cam = jnp.dot(cam_ref[...], wc_ref[...],
                  preferred_element_type=jnp.float32) + bc_ref[...]
    bi = bidx_ref[0, 0, :]
    onehot = (bi[:, None] == lax.broadcasted_iota(jnp.int32, (1, 8), 1)
              ).astype(jnp.float32)
    film = jnp.dot(onehot, cam, preferred_element_type=jnp.float32)
    scale = film[:, :_C]
    shift = film[:, _C:]
    o_ref[...] = jnp.maximum(hn * (1.0 + scale) + shift, 0.0) + x_ref[...]


def _final_stage(h, s, q, gamma, beta, cam_pad, Wc, bc, bidx3, x):
    return pl.pallas_call(
        _final_body,
        grid=(_NB,),
        in_specs=[
            pl.BlockSpec((_R, _C), lambda i: (i, 0)),
            pl.BlockSpec((1, _C), lambda i: (0, 0)),
            pl.BlockSpec((1, _C), lambda i: (0, 0)),
            pl.BlockSpec((1, _C), lambda i: (0, 0)),
            pl.BlockSpec((1, _C), lambda i: (0, 0)),
            pl.BlockSpec((8, 2 * _C), lambda i: (0, 0)),
            pl.BlockSpec((_CAM, 2 * _C), lambda i: (0, 0)),
            pl.BlockSpec((1, 2 * _C), lambda i: (0, 0)),
            pl.BlockSpec((1, 1, _R), lambda i: (i, 0, 0)),
            pl.BlockSpec((_R, _C), lambda i: (i, 0)),
        ],
        out_specs=pl.BlockSpec((_R, _C), lambda i: (i, 0)),
        out_shape=jax.ShapeDtypeStruct((_N, _C), jnp.float32),
        compiler_params=pltpu.CompilerParams(
            dimension_semantics=("arbitrary",)),
    )(h, s, q, gamma, beta, cam_pad, Wc, bc, bidx3, x)


def kernel(x_features, camera_cond, W1, b1, gamma1, beta1, W2, b2, gamma2,
           beta2, Wc, bc, edge_index, kernel_offsets, batch_idx):
    del b1, b2
    src = edge_index[0]
    dst = edge_index[1]
    g = _prep_stage(src, kernel_offsets)
    W1w = W1.transpose(1, 0, 2).reshape(_C, _K * _C).astype(jnp.bfloat16)
    W2w = W2.transpose(1, 0, 2).reshape(_C, _K * _C).astype(jnp.bfloat16)
    zrows = jnp.zeros((8, _C), jnp.float32)

    y1 = _dense_stage(x_features, W1w)
    p1 = _sc_scatter(y1.reshape(_K * _N, _C), g, dst, zrows)
    h1, s1, q1 = _stats_stage(p1.reshape(_NSC, _NPAD, _C))

    y2 = _dense_stage(h1, W2w, stats=(s1, q1, gamma1.reshape(1, _C),
                                      beta1.reshape(1, _C)))
    p2 = _sc_scatter(y2.reshape(_K * _N, _C), g, dst, zrows)
    h2, s2, q2 = _stats_stage(p2.reshape(_NSC, _NPAD, _C))

    cam_pad = jnp.zeros((8, _CAM), jnp.float32).at[:4].set(camera_cond)
    bidx3 = batch_idx.reshape(_NB, 1, _R)
    return _final_stage(h2, s2, q2, gamma2.reshape(1, _C),
                        beta2.reshape(1, _C), cam_pad, Wc,
                        bc.reshape(1, 2 * _C), bidx3, x_features)


# speedup vs baseline: 4.6535x; 1.1367x over previous
import functools

import jax
import jax.numpy as jnp
from jax import lax
from jax.experimental import pallas as pl
from jax.experimental.pallas import tpu as pltpu
from jax.experimental.pallas import tpu_sc as plsc

_N = 10000
_E = 320000
_C = 128
_K = 27
_CAM = 256
_EPS = 1e-5

_NSC = 2
_NSUB = 16
_NT = _NSC * _NSUB
_EP = _E // _NT
_CH = 64
_NCH = _EP // _CH
_CHT = _EP - _NCH * _CH
_NSLOT = 4
_NPAD = 10240
_RPT = _NPAD // _NSUB
_NB = 25
_R = _N // _NB


def _dense_body(apply_bn, x_ref, w_ref, *rest):
    if apply_bn:
        s_ref, q_ref, g_ref, b_ref, y_ref = rest
        inv_n = jnp.float32(1.0 / _N)
        mu = s_ref[...] * inv_n
        var = q_ref[...] * inv_n - mu * mu
        hn = g_ref[...] * (x_ref[...] - mu) * lax.rsqrt(var + _EPS)
        xb = jnp.maximum(hn + b_ref[...], 0.0)
    else:
        (y_ref,) = rest
        xb = x_ref[...]
    y = jnp.dot(xb.astype(jnp.bfloat16), w_ref[...],
                preferred_element_type=jnp.float32)
    for k in range(_K):
        y_ref[k] = y[:, k * _C:(k + 1) * _C]


def _dense_stage(x, Wwide, stats=None):
    apply_bn = stats is not None
    in_specs = [
        pl.BlockSpec((_R, _C), lambda i: (i, 0)),
        pl.BlockSpec((_C, _K * _C), lambda i: (0, 0)),
    ]
    args = [x, Wwide]
    if apply_bn:
        in_specs += [pl.BlockSpec((1, _C), lambda i: (0, 0))] * 4
        args += list(stats)
    return pl.pallas_call(
        functools.partial(_dense_body, apply_bn),
        grid=(_NB,),
        in_specs=in_specs,
        out_specs=pl.BlockSpec((_K, _R, _C), lambda i: (0, i, 0)),
        out_shape=jax.ShapeDtypeStruct((_K, _N, _C), jnp.float32),
        compiler_params=pltpu.CompilerParams(
            dimension_semantics=("arbitrary",)),
    )(*args)


def _prep_body(src_ref, koff_ref, g_ref):
    g_ref[...] = koff_ref[...] * _N + src_ref[...]


def _prep_stage(src, koff):
    s2 = src.reshape(_E // 128, 128)
    k2 = koff.reshape(_E // 128, 128)
    g2 = pl.pallas_call(
        _prep_body,
        out_shape=jax.ShapeDtypeStruct((_E // 128, 128), jnp.int32),
    )(s2, k2)
    return g2.reshape(_E)


def _sc_scatter(table, g, dst, zrows):
    mesh = plsc.VectorSubcoreMesh(core_axis_name="c", subcore_axis_name="s")

    scratch = (
        [pltpu.VMEM((_EP,), jnp.int32)]
        + [pltpu.VMEM((_CH,), jnp.int32)] * _NSLOT
        + [pltpu.VMEM((_CHT,), jnp.int32)]
        + [pltpu.VMEM((_CH, _C), jnp.float32)] * _NSLOT
        + [pltpu.VMEM((_CHT, _C), jnp.float32)]
        + [pltpu.VMEM((8, _C), jnp.float32)]
        + [pltpu.VMEM_SHARED((_NPAD, _C), jnp.float32)]
        + [pltpu.SemaphoreType.DMA] * (3 * _NSLOT)
    )

    @functools.partial(
        pl.kernel,
        out_type=jax.ShapeDtypeStruct((_NSC * _NPAD, _C), jnp.float32),
        mesh=mesh,
        scratch_types=scratch,
    )
    def sc_kernel(table_h, g_h, dst_h, zrows_h, out_h, g_v, *rest):
        db_v = rest[:_NSLOT]
        dbt_v = rest[_NSLOT]
        rows_v = rest[_NSLOT + 1:2 * _NSLOT + 1]
        rowt_v = rest[2 * _NSLOT + 1]
        zb_v = rest[2 * _NSLOT + 2]
        acc_s = rest[2 * _NSLOT + 3]
        gsem = rest[2 * _NSLOT + 4:3 * _NSLOT + 4]
        ssem = rest[3 * _NSLOT + 4:4 * _NSLOT + 4]
        isem = rest[4 * _NSLOT + 4:5 * _NSLOT + 4]
        cid = lax.axis_index("c")
        sid = lax.axis_index("s")
        wid = sid * _NSC + cid
        ebase = pl.multiple_of(wid * _EP, 8)

        pltpu.sync_copy(g_h.at[pl.ds(ebase, _EP)], g_v)
        pltpu.sync_copy(zrows_h, zb_v)

        rbase = sid * _RPT

        def zrow(j, c):
            ro = pl.multiple_of(rbase + j * 8, 8)
            pltpu.sync_copy(zb_v, acc_s.at[pl.ds(ro, 8)])
            return c
        lax.fori_loop(0, _RPT // 8, zrow, 0)
        plsc.subcore_barrier()

        def idx_load(i, db, sem):
            eo = pl.multiple_of(ebase + i * _CH, 8)
            return pltpu.async_copy(dst_h.at[pl.ds(eo, _CH)], db, sem)

        def idx_wait(i, db, sem):
            eo = pl.multiple_of(ebase + i * _CH, 8)
            pltpu.make_async_copy(dst_h.at[pl.ds(eo, _CH)], db, sem).wait()

        def gather(i, rows, sem):
            eb = pl.multiple_of(i * _CH, 8)
            return pltpu.async_copy(table_h.at[g_v.at[pl.ds(eb, _CH)]],
                                    rows, sem)

        def gather_wait(i, rows, sem):
            eb = pl.multiple_of(i * _CH, 8)
            pltpu.make_async_copy(table_h.at[g_v.at[pl.ds(eb, _CH)]],
                                  rows, sem).wait()

        def scatter(rows, db, sem):
            return pltpu.async_copy(rows, acc_s.at[db], sem, add=True)

        def scatter_wait(rows, db, sem):
            pltpu.make_async_copy(rows, acc_s.at[db], sem).wait()

        for s in range(_NSLOT):
            idx_load(s, db_v[s], isem[s])
            gather(s, rows_v[s], gsem[s])

        def round_(r, c):
            i0 = _NSLOT * r
            for s in range(_NSLOT):
                gather_wait(i0 + s, rows_v[s], gsem[s])
                idx_wait(i0 + s, db_v[s], isem[s])
                scatter(rows_v[s], db_v[s], ssem[s])
            for s in range(_NSLOT):
                scatter_wait(rows_v[s], db_v[s], ssem[s])
                idx_load(i0 + _NSLOT + s, db_v[s], isem[s])
                gather(i0 + _NSLOT + s, rows_v[s], gsem[s])
            return c
        lax.fori_loop(0, _NCH // _NSLOT - 1, round_, 0)

        ilast = _NCH - _NSLOT
        for s in range(_NSLOT):
            gather_wait(ilast + s, rows_v[s], gsem[s])
            idx_wait(ilast + s, db_v[s], isem[s])
            scatter(rows_v[s], db_v[s], ssem[s])
        ebt = pl.multiple_of(_NCH * _CH, 8)
        pltpu.sync_copy(dst_h.at[pl.ds(ebase + ebt, _CHT)], dbt_v)
        pltpu.async_copy(table_h.at[g_v.at[pl.ds(ebt, _CHT)]], rowt_v,
                         gsem[0]).wait()
        for s in range(_NSLOT):
            scatter_wait(rows_v[s], db_v[s], ssem[s])
        pltpu.sync_copy(rowt_v, acc_s.at[dbt_v], add=True)
        plsc.subcore_barrier()

        obase = cid * _NPAD + rbase

        def drain(j, c):
            ro = pl.multiple_of(rbase + j * 8, 8)
            oo = pl.multiple_of(obase + j * 8, 8)
            pltpu.sync_copy(acc_s.at[pl.ds(ro, 8)], zb_v)
            pltpu.sync_copy(zb_v, out_h.at[pl.ds(oo, 8)])
            return c
        lax.fori_loop(0, _RPT // 8, drain, 0)

    return sc_kernel(table, g, dst, zrows)


def _stats_body(p_ref, h_ref, sum_ref, sq_ref):
    i = pl.program_id(0)
    h = p_ref[0] + p_ref[1]
    h_ref[...] = h
    s = jnp.sum(h, axis=0, keepdims=True)
    q = jnp.sum(h * h, axis=0, keepdims=True)

    @pl.when(i == 0)
    def _():
        sum_ref[...] = s
        sq_ref[...] = q

    @pl.when(i > 0)
    def _():
        sum_ref[...] = sum_ref[...] + s
        sq_ref[...] = sq_ref[...] + q


def _stats_stage(partials):
    return pl.pallas_call(
        _stats_body,
        grid=(_NB,),
        in_specs=[pl.BlockSpec((_NSC, _R, _C), lambda i: (0, i, 0))],
        out_specs=[
            pl.BlockSpec((_R, _C), lambda i: (i, 0)),
            pl.BlockSpec((1, _C), lambda i: (0, 0)),
            pl.BlockSpec((1, _C), lambda i: (0, 0)),
        ],
        out_shape=[
            jax.ShapeDtypeStruct((_N, _C), jnp.float32),
            jax.ShapeDtypeStruct((1, _C), jnp.float32),
            jax.ShapeDtypeStruct((1, _C), jnp.float32),
        ],
        compiler_params=pltpu.CompilerParams(
            dimension_semantics=("arbitrary",)),
    )(partials)


def _final_body(h_ref, s_ref, q_ref, g_ref, b_ref, cam_ref, wc_ref, bc_ref,
                bidx_ref, x_ref, o_ref):
    inv_n = jnp.float32(1.0 / _N)
    mu = s_ref[...] * inv_n
    var = q_ref[...] * inv_n - mu * mu
    hn = g_ref[...] * (h_ref[...] - mu) * lax.rsqrt(var + _EPS) + b_ref[...]
    cam = jnp.dot(cam_ref[...], wc_ref[...],
                  preferred_element_type=jnp.float32) + bc_ref[...]
    bi = bidx_ref[0, 0, :]
    onehot = (bi[:, None] == lax.broadcasted_iota(jnp.int32, (1, 8), 1)
              ).astype(jnp.float32)
    film = jnp.dot(onehot, cam, preferred_element_type=jnp.float32)
    scale = film[:, :_C]
    shift = film[:, _C:]
    o_ref[...] = jnp.maximum(hn * (1.0 + scale) + shift, 0.0) + x_ref[...]


def _final_stage(h, s, q, gamma, beta, cam_pad, Wc, bc, bidx3, x):
    return pl.pallas_call(
        _final_body,
        grid=(_NB,),
        in_specs=[
            pl.BlockSpec((_R, _C), lambda i: (i, 0)),
            pl.BlockSpec((1, _C), lambda i: (0, 0)),
            pl.BlockSpec((1, _C), lambda i: (0, 0)),
            pl.BlockSpec((1, _C), lambda i: (0, 0)),
            pl.BlockSpec((1, _C), lambda i: (0, 0)),
            pl.BlockSpec((8, 2 * _C), lambda i: (0, 0)),
            pl.BlockSpec((_CAM, 2 * _C), lambda i: (0, 0)),
            pl.BlockSpec((1, 2 * _C), lambda i: (0, 0)),
            pl.BlockSpec((1, 1, _R), lambda i: (i, 0, 0)),
            pl.BlockSpec((_R, _C), lambda i: (i, 0)),
        ],
        out_specs=pl.BlockSpec((_R, _C), lambda i: (i, 0)),
        out_shape=jax.ShapeDtypeStruct((_N, _C), jnp.float32),
        compiler_params=pltpu.CompilerParams(
            dimension_semantics=("arbitrary",)),
    )(h, s, q, gamma, beta, cam_pad, Wc, bc, bidx3, x)


def kernel(x_features, camera_cond, W1, b1, gamma1, beta1, W2, b2, gamma2,
           beta2, Wc, bc, edge_index, kernel_offsets, batch_idx):
    del b1, b2
    src = edge_index[0]
    dst = edge_index[1]
    g = _prep_stage(src, kernel_offsets)
    W1w = W1.transpose(1, 0, 2).reshape(_C, _K * _C).astype(jnp.bfloat16)
    W2w = W2.transpose(1, 0, 2).reshape(_C, _K * _C).astype(jnp.bfloat16)
    zrows = jnp.zeros((8, _C), jnp.float32)

    y1 = _dense_stage(x_features, W1w)
    p1 = _sc_scatter(y1.reshape(_K * _N, _C), g, dst, zrows)
    h1, s1, q1 = _stats_stage(p1.reshape(_NSC, _NPAD, _C))

    y2 = _dense_stage(h1, W2w, stats=(s1, q1, gamma1.reshape(1, _C),
                                      beta1.reshape(1, _C)))
    p2 = _sc_scatter(y2.reshape(_K * _N, _C), g, dst, zrows)
    h2, s2, q2 = _stats_stage(p2.reshape(_NSC, _NPAD, _C))

    cam_pad = jnp.zeros((8, _CAM), jnp.float32).at[:4].set(camera_cond)
    bidx3 = batch_idx.reshape(_NB, 1, _R)
    return _final_stage(h2, s2, q2, gamma2.reshape(1, _C),
                        beta2.reshape(1, _C), cam_pad, Wc,
                        bc.reshape(1, 2 * _C), bidx3, x_features)


# speedup vs baseline: 5.0661x; 1.0887x over previous
import functools

import jax
import jax.numpy as jnp
from jax import lax
from jax.experimental import pallas as pl
from jax.experimental.pallas import tpu as pltpu
from jax.experimental.pallas import tpu_sc as plsc

_N = 10000
_E = 320000
_C = 128
_K = 27
_CAM = 256
_EPS = 1e-5

_NSC = 2
_NSUB = 16
_NT = _NSC * _NSUB
_EP = _E // _NT
_CH = 64
_NCH = _EP // _CH
_CHT = _EP - _NCH * _CH
_NSLOT = 4
_NPAD = 10240
_RPT = _NPAD // _NSUB
_NB = 25
_R = _N // _NB


def _dense_body(apply_bn, x_ref, w_ref, *rest):
    if apply_bn:
        s_ref, q_ref, g_ref, b_ref, y_ref = rest
        inv_n = jnp.float32(1.0 / _N)
        mu = s_ref[...] * inv_n
        var = q_ref[...] * inv_n - mu * mu
        hn = g_ref[...] * (x_ref[...] - mu) * lax.rsqrt(var + _EPS)
        xb = jnp.maximum(hn + b_ref[...], 0.0)
    else:
        (y_ref,) = rest
        xb = x_ref[...]
    y = jnp.dot(xb.astype(jnp.bfloat16), w_ref[...],
                preferred_element_type=jnp.float32)
    for k in range(_K):
        y_ref[k] = y[:, k * _C:(k + 1) * _C]


def _dense_stage(x, Wwide, stats=None):
    apply_bn = stats is not None
    in_specs = [
        pl.BlockSpec((_R, _C), lambda i: (i, 0)),
        pl.BlockSpec((_C, _K * _C), lambda i: (0, 0)),
    ]
    args = [x, Wwide]
    if apply_bn:
        in_specs += [pl.BlockSpec((1, _C), lambda i: (0, 0))] * 4
        args += list(stats)
    return pl.pallas_call(
        functools.partial(_dense_body, apply_bn),
        grid=(_NB,),
        in_specs=in_specs,
        out_specs=pl.BlockSpec((_K, _R, _C), lambda i: (0, i, 0)),
        out_shape=jax.ShapeDtypeStruct((_K, _N, _C), jnp.float32),
        compiler_params=pltpu.CompilerParams(
            dimension_semantics=("arbitrary",)),
    )(*args)


def _prep_body(src_ref, koff_ref, g_ref):
    g_ref[...] = koff_ref[...] * _N + src_ref[...]


def _prep_stage(src, koff):
    s2 = src.reshape(_E // 128, 128)
    k2 = koff.reshape(_E // 128, 128)
    g2 = pl.pallas_call(
        _prep_body,
        out_shape=jax.ShapeDtypeStruct((_E // 128, 128), jnp.int32),
    )(s2, k2)
    return g2.reshape(_E)


def _sc_scatter(table, g, dst, zrows):
    mesh = plsc.VectorSubcoreMesh(core_axis_name="c", subcore_axis_name="s")

    scratch = (
        [pltpu.VMEM((_EP,), jnp.int32)]
        + [pltpu.VMEM((_CH,), jnp.int32)] * _NSLOT
        + [pltpu.VMEM((_CHT,), jnp.int32)]
        + [pltpu.VMEM((_CH, _C), jnp.float32)] * _NSLOT
        + [pltpu.VMEM((32, _C), jnp.float32)]
        + [pltpu.VMEM_SHARED((_NPAD, _C), jnp.float32)]
        + [pltpu.SemaphoreType.DMA] * (3 * _NSLOT)
    )

    @functools.partial(
        pl.kernel,
        out_type=jax.ShapeDtypeStruct((_NSC * _NPAD, _C), jnp.float32),
        mesh=mesh,
        scratch_types=scratch,
    )
    def sc_kernel(table_h, g_h, dst_h, zrows_h, out_h, g_v, *rest):
        db_v = rest[:_NSLOT]
        dbt_v = rest[_NSLOT]
        rows_v = rest[_NSLOT + 1:2 * _NSLOT + 1]
        zb_v = rest[2 * _NSLOT + 1]
        acc_s = rest[2 * _NSLOT + 2]
        gsem = rest[2 * _NSLOT + 3:3 * _NSLOT + 3]
        ssem = rest[3 * _NSLOT + 3:4 * _NSLOT + 3]
        isem = rest[4 * _NSLOT + 3:5 * _NSLOT + 3]
        cid = lax.axis_index("c")
        sid = lax.axis_index("s")
        wid = sid * _NSC + cid
        ebase = pl.multiple_of(wid * _EP, 8)

        pltpu.sync_copy(g_h.at[pl.ds(ebase, _EP)], g_v)
        pltpu.sync_copy(zrows_h, zb_v)

        rbase = sid * _RPT
        for j in range(_RPT // 32):
            ro = pl.multiple_of(rbase + j * 32, 8)
            pltpu.async_copy(zb_v, acc_s.at[pl.ds(ro, 32)], gsem[0])
        for j in range(_RPT // 32):
            ro = pl.multiple_of(rbase + j * 32, 8)
            pltpu.make_async_copy(zb_v, acc_s.at[pl.ds(ro, 32)],
                                  gsem[0]).wait()
        plsc.subcore_barrier()

        def idx_load(i, db, sem):
            eo = pl.multiple_of(ebase + i * _CH, 8)
            return pltpu.async_copy(dst_h.at[pl.ds(eo, _CH)], db, sem)

        def idx_wait(i, db, sem):
            eo = pl.multiple_of(ebase + i * _CH, 8)
            pltpu.make_async_copy(dst_h.at[pl.ds(eo, _CH)], db, sem).wait()

        def gather(i, rows, sem):
            eb = pl.multiple_of(i * _CH, 8)
            return pltpu.async_copy(table_h.at[g_v.at[pl.ds(eb, _CH)]],
                                    rows, sem)

        def gather_wait(i, rows, sem):
            eb = pl.multiple_of(i * _CH, 8)
            pltpu.make_async_copy(table_h.at[g_v.at[pl.ds(eb, _CH)]],
                                  rows, sem).wait()

        def scatter(rows, db, sem):
            return pltpu.async_copy(rows, acc_s.at[db], sem, add=True)

        def scatter_wait(rows, db, sem):
            pltpu.make_async_copy(rows, acc_s.at[db], sem).wait()

        for s in range(_NSLOT):
            idx_load(s, db_v[s], isem[s])
            gather(s, rows_v[s], gsem[s])

        def round_(r, c):
            i0 = _NSLOT * r
            for s in range(_NSLOT):
                gather_wait(i0 + s, rows_v[s], gsem[s])
                idx_wait(i0 + s, db_v[s], isem[s])
                scatter(rows_v[s], db_v[s], ssem[s])
            for s in range(_NSLOT):
                scatter_wait(rows_v[s], db_v[s], ssem[s])
                idx_load(i0 + _NSLOT + s, db_v[s], isem[s])
                gather(i0 + _NSLOT + s, rows_v[s], gsem[s])
            return c
        lax.fori_loop(0, _NCH // _NSLOT - 1, round_, 0)

        ilast = _NCH - _NSLOT
        for s in range(_NSLOT):
            gather_wait(ilast + s, rows_v[s], gsem[s])
            idx_wait(ilast + s, db_v[s], isem[s])
            scatter(rows_v[s], db_v[s], ssem[s])
        ebt = pl.multiple_of(_NCH * _CH, 8)
        pltpu.sync_copy(dst_h.at[pl.ds(ebase + ebt, _CHT)], dbt_v)
        for s in range(_NSLOT):
            scatter_wait(rows_v[s], db_v[s], ssem[s])
        pltpu.async_copy(table_h.at[g_v.at[pl.ds(ebt, _CHT)]],
                         rows_v[0].at[pl.ds(0, _CHT)], gsem[0]).wait()
        pltpu.sync_copy(rows_v[0].at[pl.ds(0, _CHT)], acc_s.at[dbt_v],
                        add=True)
        plsc.subcore_barrier()

        obase = cid * _NPAD + rbase

        def d_in(j, s):
            ro = pl.multiple_of(rbase + j * 64, 8)
            return pltpu.async_copy(acc_s.at[pl.ds(ro, 64)], rows_v[s],
                                    gsem[s])

        def d_out(j, s):
            oo = pl.multiple_of(obase + j * 64, 8)
            return pltpu.async_copy(rows_v[s], out_h.at[pl.ds(oo, 64)],
                                    ssem[s])

        def d_out_wait(j, s):
            oo = pl.multiple_of(obase + j * 64, 8)
            pltpu.make_async_copy(rows_v[s], out_h.at[pl.ds(oo, 64)],
                                  ssem[s]).wait()

        nd = _RPT // 64
        for j in range(nd):
            s = j % 2
            if j >= 2:
                d_out_wait(j - 2, s)
            d_in(j, s).wait()
            d_out(j, s)
        d_out_wait(nd - 2, (nd - 2) % 2)
        d_out_wait(nd - 1, (nd - 1) % 2)

    return sc_kernel(table, g, dst, zrows)


def _stats_body(p_ref, h_ref, sum_ref, sq_ref):
    i = pl.program_id(0)
    h = p_ref[0] + p_ref[1]
    h_ref[...] = h
    s = jnp.sum(h, axis=0, keepdims=True)
    q = jnp.sum(h * h, axis=0, keepdims=True)

    @pl.when(i == 0)
    def _():
        sum_ref[...] = s
        sq_ref[...] = q

    @pl.when(i > 0)
    def _():
        sum_ref[...] = sum_ref[...] + s
        sq_ref[...] = sq_ref[...] + q


def _stats_stage(partials):
    return pl.pallas_call(
        _stats_body,
        grid=(_NB,),
        in_specs=[pl.BlockSpec((_NSC, _R, _C), lambda i: (0, i, 0))],
        out_specs=[
            pl.BlockSpec((_R, _C), lambda i: (i, 0)),
            pl.BlockSpec((1, _C), lambda i: (0, 0)),
            pl.BlockSpec((1, _C), lambda i: (0, 0)),
        ],
        out_shape=[
            jax.ShapeDtypeStruct((_N, _C), jnp.float32),
            jax.ShapeDtypeStruct((1, _C), jnp.float32),
            jax.ShapeDtypeStruct((1, _C), jnp.float32),
        ],
        compiler_params=pltpu.CompilerParams(
            dimension_semantics=("arbitrary",)),
    )(partials)


def _final_body(h_ref, s_ref, q_ref, g_ref, b_ref, cam_ref, wc_ref, bc_ref,
                bidx_ref, x_ref, o_ref):
    inv_n = jnp.float32(1.0 / _N)
    mu = s_ref[...] * inv_n
    var = q_ref[...] * inv_n - mu * mu
    hn = g_ref[...] * (h_ref[...] - mu) * lax.rsqrt(var + _EPS) + b_ref[...]
    cam = jnp.dot(cam_ref[...], wc_ref[...],
                  preferred_element_type=jnp.float32) + bc_ref[...]
    bi = bidx_ref[0, 0, :]
    onehot = (bi[:, None] == lax.broadcasted_iota(jnp.int32, (1, 8), 1)
              ).astype(jnp.float32)
    film = jnp.dot(onehot, cam, preferred_element_type=jnp.float32)
    scale = film[:, :_C]
    shift = film[:, _C:]
    o_ref[...] = jnp.maximum(hn * (1.0 + scale) + shift, 0.0) + x_ref[...]


def _final_stage(h, s, q, gamma, beta, cam_pad, Wc, bc, bidx3, x):
    return pl.pallas_call(
        _final_body,
        grid=(_NB,),
        in_specs=[
            pl.BlockSpec((_R, _C), lambda i: (i, 0)),
            pl.BlockSpec((1, _C), lambda i: (0, 0)),
            pl.BlockSpec((1, _C), lambda i: (0, 0)),
            pl.BlockSpec((1, _C), lambda i: (0, 0)),
            pl.BlockSpec((1, _C), lambda i: (0, 0)),
            pl.BlockSpec((8, 2 * _C), lambda i: (0, 0)),
            pl.BlockSpec((_CAM, 2 * _C), lambda i: (0, 0)),
            pl.BlockSpec((1, 2 * _C), lambda i: (0, 0)),
            pl.BlockSpec((1, 1, _R), lambda i: (i, 0, 0)),
            pl.BlockSpec((_R, _C), lambda i: (i, 0)),
        ],
        out_specs=pl.BlockSpec((_R, _C), lambda i: (i, 0)),
        out_shape=jax.ShapeDtypeStruct((_N, _C), jnp.float32),
        compiler_params=pltpu.CompilerParams(
            dimension_semantics=("arbitrary",)),
    )(h, s, q, gamma, beta, cam_pad, Wc, bc, bidx3, x)


def kernel(x_features, camera_cond, W1, b1, gamma1, beta1, W2, b2, gamma2,
           beta2, Wc, bc, edge_index, kernel_offsets, batch_idx):
    del b1, b2
    src = edge_index[0]
    dst = edge_index[1]
    g = _prep_stage(src, kernel_offsets)
    W1w = W1.transpose(1, 0, 2).reshape(_C, _K * _C).astype(jnp.bfloat16)
    W2w = W2.transpose(1, 0, 2).reshape(_C, _K * _C).astype(jnp.bfloat16)
    zrows = jnp.zeros((32, _C), jnp.float32)

    y1 = _dense_stage(x_features, W1w)
    p1 = _sc_scatter(y1.reshape(_K * _N, _C), g, dst, zrows)
    h1, s1, q1 = _stats_stage(p1.reshape(_NSC, _NPAD, _C))

    y2 = _dense_stage(h1, W2w, stats=(s1, q1, gamma1.reshape(1, _C),
                                      beta1.reshape(1, _C)))
    p2 = _sc_scatter(y2.reshape(_K * _N, _C), g, dst, zrows)
    h2, s2, q2 = _stats_stage(p2.reshape(_NSC, _NPAD, _C))

    cam_pad = jnp.zeros((8, _CAM), jnp.float32).at[:4].set(camera_cond)
    bidx3 = batch_idx.reshape(_NB, 1, _R)
    return _final_stage(h2, s2, q2, gamma2.reshape(1, _C),
                        beta2.reshape(1, _C), cam_pad, Wc,
                        bc.reshape(1, 2 * _C), bidx3, x_features)
